# Initial kernel scaffold; baseline (speedup 1.0000x reference)
import functools

import jax
import jax.numpy as jnp
from jax import lax
from jax.experimental import pallas as pl
from jax.experimental.pallas import tpu as pltpu
from jax.experimental.pallas import tpu_sc as plsc

NN = 10000
EE = 320000
NC = 2
NS = 16
NW = NC * NS
EW = EE // NW
L = 16

N4P = 40960
SEG1 = N4P // NS
NP = 10240
SEG2 = NP // NS
K1 = 400
K2 = 80
CP = 48

_EPS = jnp.float32(1e-16)


def _tc_a_body(x_ref, w1_ref, asm_ref, adm_ref, h1_ref, as_ref, ad_ref, c0_ref):
    h1 = jnp.dot(x_ref[...], w1_ref[...], preferred_element_type=jnp.float32)
    h1_ref[...] = h1
    a_s = jnp.dot(h1, asm_ref[...], preferred_element_type=jnp.float32)
    a_d = jnp.dot(h1, adm_ref[...], preferred_element_type=jnp.float32)
    as_ref[...] = a_s
    ad_ref[...] = a_d
    c0 = jnp.max(a_s, axis=0, keepdims=True) + jnp.max(a_d, axis=0, keepdims=True)
    c0_ref[...] = jnp.maximum(c0, 0.0)


def _tc_b_body(p_ref, b1_ref, w2_ref, a2s_ref, a2d_ref,
               emb_ref, h2_ref, as2_ref, ad2_ref, c02_ref):
    h1 = p_ref[0] + p_ref[1] + b1_ref[...]
    emb = jnp.where(h1 > 0, h1, jnp.expm1(jnp.minimum(h1, 0.0)))
    emb_ref[...] = emb
    h2 = jnp.dot(emb, w2_ref[...], preferred_element_type=jnp.float32)
    h2_ref[...] = h2
    as2 = jnp.dot(h2, a2s_ref[...], preferred_element_type=jnp.float32)
    ad2 = jnp.dot(h2, a2d_ref[...], preferred_element_type=jnp.float32)
    as2_ref[...] = as2
    ad2_ref[...] = ad2
    c02 = jnp.max(as2, axis=0, keepdims=True) + jnp.max(ad2, axis=0, keepdims=True)
    c02_ref[...] = jnp.maximum(c02, 0.0)


def _tc_c_body(p_ref, b2_ref, out_ref):
    o = p_ref[0] + p_ref[1] + b2_ref[...]
    mask = lax.broadcasted_iota(jnp.int32, (NN, CP), 1) < 40
    xm = jnp.where(mask, o, jnp.float32(-1e30))
    m = jnp.max(xm, axis=1, keepdims=True)
    ex = jnp.where(mask, jnp.exp(o - m), 0.0)
    lse = jnp.log(jnp.sum(ex, axis=1, keepdims=True))
    out_ref[...] = o - m - lse


def _worker_id():
    return lax.axis_index("s") * NC + lax.axis_index("c")


def _vadd_loop(dst_ref, src_ref, dst_off, n_vregs):
    def body(i, _):
        o = dst_off + i * L
        dst_ref[pl.ds(o, L)] = dst_ref[pl.ds(o, L)] + src_ref[pl.ds(i * L, L)]
        return _
    lax.fori_loop(0, n_vregs, body, None)


def _zero_loop(dst_ref, n_vregs):
    z = jnp.zeros((L,), jnp.float32)
    def body(i, _):
        dst_ref[pl.ds(i * L, L)] = z
        return _
    lax.fori_loop(0, n_vregs, body, None)


def _l1p1_body(src_h, dst_h, asT_h, adT_h, c0_h,
               den_h, ex_h,
               as_v, ad_v, den_v, c0_v, sv, dv, exv, acc_v, tmp_v, den_sh):
    cid = lax.axis_index("c")
    sid = lax.axis_index("s")
    wid = _worker_id()
    lidx = lax.iota(jnp.int32, L)
    for g in range(2):
        pltpu.sync_copy(asT_h.at[g], as_v)
        pltpu.sync_copy(adT_h.at[g], ad_v)
        pltpu.sync_copy(c0_h.at[g], c0_v)
        c0vec = c0_v[...]
        _zero_loop(den_v, N4P // L)

        def chunk(i, _):
            base = wid * EW + i * K1
            pltpu.sync_copy(src_h.at[pl.ds(base, K1)], sv)
            pltpu.sync_copy(dst_h.at[pl.ds(base, K1)], dv)

            def step(j, _):
                epos = j * 4 + (lidx >> 2)
                s4 = plsc.load_gather(sv, [epos])
                d4 = plsc.load_gather(dv, [epos])
                his = s4 * 4 + (lidx & 3)
                hdd = d4 * 4 + (lidx & 3)
                a = plsc.load_gather(as_v, [his])
                b = plsc.load_gather(ad_v, [hdd])
                t = a + b
                e = jnp.where(t > 0, t, t * jnp.float32(0.2))
                ex = jnp.exp(e - c0vec)
                plsc.addupdate_scatter(den_v, [hdd], ex)
                exv[pl.ds(j * L, L)] = ex
                return _

            lax.fori_loop(0, K1 // 4, step, None)
            pltpu.sync_copy(exv, ex_h.at[g, pl.ds(base * 4, K1 * 4)])
            return _

        lax.fori_loop(0, EW // K1, chunk, None)

        pltpu.sync_copy(den_v, den_sh.at[sid])
        plsc.subcore_barrier()
        pltpu.sync_copy(den_sh.at[0, pl.ds(sid * SEG1, SEG1)], acc_v)
        for t in range(1, NS):
            pltpu.sync_copy(den_sh.at[t, pl.ds(sid * SEG1, SEG1)], tmp_v)
            _vadd_loop(acc_v, tmp_v, 0, SEG1 // L)
        pltpu.sync_copy(acc_v, den_h.at[cid, g, pl.ds(sid * SEG1, SEG1)])
        plsc.subcore_barrier()


def _l1p2_body(src_h, dst_h, ex_h, den_h, h1_h,
               outp_h,
               den0_v, den1_v, tmp_v, sv, dv, ex0_v, ex1_v, albuf,
               hbuf, obuf, sem, out_sh):
    cid = lax.axis_index("c")
    sid = lax.axis_index("s")
    wid = _worker_id()
    lidx = lax.iota(jnp.int32, L)

    for g, dgv in ((0, den0_v), (1, den1_v)):
        pltpu.sync_copy(den_h.at[0, g], dgv)
        for t in range(NS):
            pltpu.sync_copy(den_h.at[1, g, pl.ds(t * SEG1, SEG1)], tmp_v)
            _vadd_loop(dgv, tmp_v, t * SEG1, SEG1 // L)

    _zero_loop(obuf, K2 * 64 // L)
    rows = NN // NS
    stripe = sid * rows
    for r in range(0, rows - K2 + 1, K2):
        pltpu.sync_copy(obuf, out_sh.at[pl.ds(stripe + r, K2)])
    tail = rows % K2
    if tail:
        pltpu.sync_copy(obuf.at[pl.ds(0, tail)],
                        out_sh.at[pl.ds(stripe + rows - tail, tail)])
    plsc.subcore_barrier()

    def chunk(i, _):
        base = wid * EW + i * K2
        pltpu.sync_copy(src_h.at[pl.ds(base, K2)], sv)
        pltpu.sync_copy(dst_h.at[pl.ds(base, K2)], dv)
        pltpu.async_copy(h1_h.at[sv], hbuf, sem).wait()
        pltpu.sync_copy(ex_h.at[0, pl.ds(base * 4, K2 * 4)], ex0_v)
        pltpu.sync_copy(ex_h.at[1, pl.ds(base * 4, K2 * 4)], ex1_v)

        for g, exg, dgv in ((0, ex0_v, den0_v), (1, ex1_v, den1_v)):
            def astep(j, _, exg=exg, dgv=dgv, g=g):
                ex16 = exg[pl.ds(j * L, L)]
                d4 = plsc.load_gather(dv, [j * 4 + (lidx >> 2)])
                hdd = d4 * 4 + (lidx & 3)
                den16 = plsc.load_gather(dgv, [hdd])
                al = ex16 / (den16 + _EPS)
                aidx = (j * 4 + (lidx >> 2)) * 8 + g * 4 + (lidx & 3)
                plsc.store_scatter(albuf, [aidx], al)
                return _
            lax.fori_loop(0, K2 // 4, astep, None)

        def mstep(e, _):
            for q in range(4):
                av = plsc.load_gather(albuf, [e * 8 + q * 2 + (lidx >> 3)])
                obuf[e, pl.ds(q * L, L)] = av * hbuf[e, pl.ds(q * L, L)]
            return _
        lax.fori_loop(0, K2, mstep, None)

        pltpu.sync_copy(obuf, out_sh.at[dv], add=True)
        return _

    lax.fori_loop(0, EW // K2, chunk, None)
    plsc.subcore_barrier()
    pltpu.sync_copy(out_sh.at[pl.ds(stripe, rows)],
                    outp_h.at[cid, pl.ds(stripe, rows)])


def _l2p1_body(src_h, dst_h, as_h, ad_h, c0_h,
               den_h, ex_h,
               as_v, ad_v, den_v, c0_v, sv, dv, exv, acc_v, tmp_v, den_sh):
    cid = lax.axis_index("c")
    sid = lax.axis_index("s")
    wid = _worker_id()
    pltpu.sync_copy(as_h, as_v)
    pltpu.sync_copy(ad_h, ad_v)
    pltpu.sync_copy(c0_h, c0_v)
    c0vec = c0_v[...]
    _zero_loop(den_v, NP // L)

    def chunk(i, _):
        base = wid * EW + i * K1
        pltpu.sync_copy(src_h.at[pl.ds(base, K1)], sv)
        pltpu.sync_copy(dst_h.at[pl.ds(base, K1)], dv)

        def step(j, _):
            s16 = sv[pl.ds(j * L, L)]
            d16 = dv[pl.ds(j * L, L)]
            a = plsc.load_gather(as_v, [s16])
            b = plsc.load_gather(ad_v, [d16])
            t = a + b
            e = jnp.where(t > 0, t, t * jnp.float32(0.2))
            ex = jnp.exp(e - c0vec)
            plsc.addupdate_scatter(den_v, [d16], ex)
            exv[pl.ds(j * L, L)] = ex
            return _

        lax.fori_loop(0, K1 // L, step, None)
        pltpu.sync_copy(exv, ex_h.at[pl.ds(base, K1)])
        return _

    lax.fori_loop(0, EW // K1, chunk, None)

    pltpu.sync_copy(den_v, den_sh.at[sid])
    plsc.subcore_barrier()
    pltpu.sync_copy(den_sh.at[0, pl.ds(sid * SEG2, SEG2)], acc_v)
    for t in range(1, NS):
        pltpu.sync_copy(den_sh.at[t, pl.ds(sid * SEG2, SEG2)], tmp_v)
        _vadd_loop(acc_v, tmp_v, 0, SEG2 // L)
    pltpu.sync_copy(acc_v, den_h.at[cid, pl.ds(sid * SEG2, SEG2)])


def _l2p2_body(src_h, dst_h, ex_h, den_h, h2_h,
               outp_h,
               den_v, tmp_v, sv, dv, exv, albuf, hbuf, obuf, sem, out_sh):
    cid = lax.axis_index("c")
    sid = lax.axis_index("s")
    wid = _worker_id()
    lidx = lax.iota(jnp.int32, L)

    pltpu.sync_copy(den_h.at[0], den_v)
    for t in range(NS):
        pltpu.sync_copy(den_h.at[1, pl.ds(t * SEG2, SEG2)], tmp_v)
        _vadd_loop(den_v, tmp_v, t * SEG2, SEG2 // L)

    _zero_loop(obuf, K2 * CP // L)
    rows = NN // NS
    stripe = sid * rows
    for r in range(0, rows - K2 + 1, K2):
        pltpu.sync_copy(obuf, out_sh.at[pl.ds(stripe + r, K2)])
    tail = rows % K2
    if tail:
        pltpu.sync_copy(obuf.at[pl.ds(0, tail)],
                        out_sh.at[pl.ds(stripe + rows - tail, tail)])
    plsc.subcore_barrier()

    def chunk(i, _):
        base = wid * EW + i * K2
        pltpu.sync_copy(src_h.at[pl.ds(base, K2)], sv)
        pltpu.sync_copy(dst_h.at[pl.ds(base, K2)], dv)
        pltpu.async_copy(h2_h.at[sv], hbuf, sem).wait()
        pltpu.sync_copy(ex_h.at[pl.ds(base, K2)], exv)

        def astep(j, _):
            d16 = dv[pl.ds(j * L, L)]
            den16 = plsc.load_gather(den_v, [d16])
            al = exv[pl.ds(j * L, L)] / (den16 + _EPS)
            albuf[pl.ds(j * L, L)] = al
            return _
        lax.fori_loop(0, K2 // L, astep, None)

        def mstep(e, _):
            av = plsc.load_gather(albuf, [lidx * 0 + e])
            for q in range(CP // L):
                obuf[e, pl.ds(q * L, L)] = av * hbuf[e, pl.ds(q * L, L)]
            return _
        lax.fori_loop(0, K2, mstep, None)

        pltpu.sync_copy(obuf, out_sh.at[dv], add=True)
        return _

    lax.fori_loop(0, EW // K2, chunk, None)
    plsc.subcore_barrier()
    pltpu.sync_copy(out_sh.at[pl.ds(stripe, rows)],
                    outp_h.at[cid, pl.ds(stripe, rows)])


def _sc_mesh():
    return plsc.VectorSubcoreMesh(
        core_axis_name="c", subcore_axis_name="s", num_cores=NC, num_subcores=NS)


def kernel(x, edge_index, W1, att_src1, att_dst1, b1, W2, att_src2, att_dst2, b2):
    f32 = jnp.float32
    src = edge_index[0]
    dst = edge_index[1]

    asf = att_src1.reshape(64)
    adf = att_dst1.reshape(64)
    hm = (jnp.arange(64)[:, None] // 8 == jnp.arange(8)[None, :]).astype(f32)
    asmask = hm * asf[:, None]
    admask = hm * adf[:, None]

    h1, a_s, a_d, c0 = pl.pallas_call(
        _tc_a_body,
        out_shape=(
            jax.ShapeDtypeStruct((NN, 64), f32),
            jax.ShapeDtypeStruct((NN, 8), f32),
            jax.ShapeDtypeStruct((NN, 8), f32),
            jax.ShapeDtypeStruct((1, 8), f32),
        ),
    )(x, W1, asmask, admask)

    def to_groups(a):
        g = a.reshape(NN, 2, 4).transpose(1, 0, 2).reshape(2, NN * 4)
        return jnp.pad(g, ((0, 0), (0, N4P - NN * 4)))

    asT = to_groups(a_s)
    adT = to_groups(a_d)
    c0dup = jnp.tile(c0.reshape(2, 4), (1, 4))

    mesh = _sc_mesh()

    den1, ex1 = pl.kernel(
        _l1p1_body, mesh=mesh,
        out_type=(
            jax.ShapeDtypeStruct((NC, 2, N4P), f32),
            jax.ShapeDtypeStruct((2, EE * 4), f32),
        ),
        scratch_types=[
            pltpu.VMEM((N4P,), f32),
            pltpu.VMEM((N4P,), f32),
            pltpu.VMEM((N4P,), f32),
            pltpu.VMEM((L,), f32),
            pltpu.VMEM((K1,), jnp.int32),
            pltpu.VMEM((K1,), jnp.int32),
            pltpu.VMEM((K1 * 4,), f32),
            pltpu.VMEM((SEG1,), f32),
            pltpu.VMEM((SEG1,), f32),
            pltpu.VMEM_SHARED((NS, N4P), f32),
        ],
    )(src, dst, asT, adT, c0dup)

    outp1 = pl.kernel(
        _l1p2_body, mesh=mesh,
        out_type=jax.ShapeDtypeStruct((NC, NN, 64), f32),
        scratch_types=[
            pltpu.VMEM((N4P,), f32),
            pltpu.VMEM((N4P,), f32),
            pltpu.VMEM((SEG1,), f32),
            pltpu.VMEM((K2,), jnp.int32),
            pltpu.VMEM((K2,), jnp.int32),
            pltpu.VMEM((K2 * 4,), f32),
            pltpu.VMEM((K2 * 4,), f32),
            pltpu.VMEM((K2 * 8,), f32),
            pltpu.VMEM((K2, 64), f32),
            pltpu.VMEM((K2, 64), f32),
            pltpu.SemaphoreType.DMA,
            pltpu.VMEM_SHARED((NN, 64), f32),
        ],
    )(src, dst, ex1, den1, h1)

    w2p = jnp.pad(W2, ((0, 0), (0, CP - 40)))
    a2sp = jnp.pad(att_src2.reshape(40, 1), ((0, CP - 40), (0, 0)))
    a2dp = jnp.pad(att_dst2.reshape(40, 1), ((0, CP - 40), (0, 0)))
    emb, h2, as2, ad2, c02 = pl.pallas_call(
        _tc_b_body,
        out_shape=(
            jax.ShapeDtypeStruct((NN, 64), f32),
            jax.ShapeDtypeStruct((NN, CP), f32),
            jax.ShapeDtypeStruct((NN, 1), f32),
            jax.ShapeDtypeStruct((NN, 1), f32),
            jax.ShapeDtypeStruct((1, 1), f32),
        ),
    )(outp1, b1.reshape(1, 64), w2p, a2sp, a2dp)

    as2p = jnp.pad(as2.reshape(NN), (0, NP - NN))
    ad2p = jnp.pad(ad2.reshape(NN), (0, NP - NN))
    c02dup = jnp.broadcast_to(c02.reshape(1), (L,))

    den2, ex2 = pl.kernel(
        _l2p1_body, mesh=mesh,
        out_type=(
            jax.ShapeDtypeStruct((NC, NP), f32),
            jax.ShapeDtypeStruct((EE,), f32),
        ),
        scratch_types=[
            pltpu.VMEM((NP,), f32),
            pltpu.VMEM((NP,), f32),
            pltpu.VMEM((NP,), f32),
            pltpu.VMEM((L,), f32),
            pltpu.VMEM((K1,), jnp.int32),
            pltpu.VMEM((K1,), jnp.int32),
            pltpu.VMEM((K1,), f32),
            pltpu.VMEM((SEG2,), f32),
            pltpu.VMEM((SEG2,), f32),
            pltpu.VMEM_SHARED((NS, NP), f32),
        ],
    )(src, dst, as2p, ad2p, c02dup)

    outp2 = pl.kernel(
        _l2p2_body, mesh=mesh,
        out_type=jax.ShapeDtypeStruct((NC, NN, CP), f32),
        scratch_types=[
            pltpu.VMEM((NP,), f32),
            pltpu.VMEM((SEG2,), f32),
            pltpu.VMEM((K2,), jnp.int32),
            pltpu.VMEM((K2,), jnp.int32),
            pltpu.VMEM((K2,), f32),
            pltpu.VMEM((K2,), f32),
            pltpu.VMEM((K2, CP), f32),
            pltpu.VMEM((K2, CP), f32),
            pltpu.SemaphoreType.DMA,
            pltpu.VMEM_SHARED((NN, CP), f32),
        ],
    )(src, dst, ex2, den2, h2)

    b2p = jnp.pad(b2, (0, CP - 40)).reshape(1, CP)
    out48 = pl.pallas_call(
        _tc_c_body,
        out_shape=jax.ShapeDtypeStruct((NN, CP), f32),
    )(outp2, b2p)

    return (out48[:, :40], emb)


# speedup vs baseline: 27.5734x; 27.5734x over previous
import functools

import jax
import jax.numpy as jnp
from jax import lax
from jax.experimental import pallas as pl
from jax.experimental.pallas import tpu as pltpu
from jax.experimental.pallas import tpu_sc as plsc

NN = 10000
EE = 320000
NC = 2
NS = 16
NW = NC * NS
EW = EE // NW
L = 16

N4P = 40960
SEG1 = N4P // NW
NP = 10240
SEG2 = NP // NS
K1 = 400
K2 = 80
CP = 48
RT = 624
RTB = NS * RT
REX = NN - RTB

_EPS = 1e-16


def _tc_a_body(x_ref, w1_ref, asm_ref, adm_ref, h1_ref, as_ref, ad_ref, c0_ref):
    h1 = jnp.dot(x_ref[...], w1_ref[...], preferred_element_type=jnp.float32)
    h1_ref[...] = h1
    a_s = jnp.dot(h1, asm_ref[...], preferred_element_type=jnp.float32)
    a_d = jnp.dot(h1, adm_ref[...], preferred_element_type=jnp.float32)
    as_ref[...] = a_s
    ad_ref[...] = a_d
    c0 = jnp.max(a_s, axis=0, keepdims=True) + jnp.max(a_d, axis=0, keepdims=True)
    c0_ref[...] = jnp.maximum(c0, 0.0)


def _tc_b_body(pa_ref, pb_ref, b1_ref, w2_ref, a2s_ref, a2d_ref,
               emb_ref, h2_ref, as2_ref, ad2_ref, c02_ref):
    ha = pa_ref[0] + pa_ref[1]
    hb = pb_ref[0] + pb_ref[1]
    h1 = jnp.concatenate([ha, hb], axis=1) + b1_ref[...]
    emb = jnp.where(h1 > 0, h1, jnp.exp(jnp.minimum(h1, 0.0)) - 1.0)
    emb_ref[...] = emb
    h2 = jnp.dot(emb, w2_ref[...], preferred_element_type=jnp.float32)
    h2_ref[...] = h2
    as2 = jnp.dot(h2, a2s_ref[...], preferred_element_type=jnp.float32)
    ad2 = jnp.dot(h2, a2d_ref[...], preferred_element_type=jnp.float32)
    as2_ref[...] = as2
    ad2_ref[...] = ad2
    c02 = jnp.max(as2, axis=0, keepdims=True) + jnp.max(ad2, axis=0, keepdims=True)
    c02_ref[...] = jnp.maximum(c02, 0.0)


def _tc_c_body(p_ref, b2_ref, out_ref):
    o = p_ref[0] + p_ref[1] + b2_ref[...]
    mask = lax.broadcasted_iota(jnp.int32, (NN, CP), 1) < 40
    xm = jnp.where(mask, o, jnp.float32(-1e30))
    m = jnp.max(xm, axis=1, keepdims=True)
    ex = jnp.where(mask, jnp.exp(o - m), 0.0)
    lse = jnp.log(jnp.sum(ex, axis=1, keepdims=True))
    out_ref[...] = o - m - lse


def _worker_id():
    return lax.axis_index("s") * NC + lax.axis_index("c")


def _vadd_loop(dst_ref, src_ref, dst_off, n_vregs):
    def body(i, _):
        o = dst_off + i * L
        dst_ref[pl.ds(o, L)] = dst_ref[pl.ds(o, L)] + src_ref[pl.ds(i * L, L)]
        return _
    lax.fori_loop(0, n_vregs, body, None)


def _zero_loop(dst_ref, n_vregs):
    z = jnp.zeros((L,), jnp.float32)
    def body(i, _):
        dst_ref[pl.ds(i * L, L)] = z
        return _
    lax.fori_loop(0, n_vregs, body, None)


def _zero2d(dst_ref, nrows, ncols):
    z = jnp.zeros((L,), jnp.float32)
    def body(r, _):
        for q in range(ncols // L):
            dst_ref[r, pl.ds(q * L, L)] = z
        return _
    lax.fori_loop(0, nrows, body, None)


def _zero_out_shared(out_sh, obuf, sid):
    stripe = sid * RT
    for r in range(0, RT - K2 + 1, K2):
        pltpu.sync_copy(obuf, out_sh.at[pl.ds(stripe + r, K2)])
    pltpu.sync_copy(obuf.at[pl.ds(0, RT - 7 * K2)],
                    out_sh.at[pl.ds(stripe + 7 * K2, RT - 7 * K2)])

    @pl.when(sid == 0)
    def _():
        pltpu.sync_copy(obuf.at[pl.ds(0, REX)], out_sh.at[pl.ds(RTB, REX)])


def _write_out_shared(out_sh, outp_h, cid, sid):
    stripe = sid * RT
    pltpu.sync_copy(out_sh.at[pl.ds(stripe, RT)],
                    outp_h.at[cid, pl.ds(stripe, RT)])

    @pl.when(sid == 0)
    def _():
        pltpu.sync_copy(out_sh.at[pl.ds(RTB, REX)],
                        outp_h.at[cid, pl.ds(RTB, REX)])


def _l1p1_body(src_h, dst_h, asT_h, adT_h, c0_h,
               denp_h, ex_h,
               as_v, ad_v, den_v, c0_v, sv, dv, exv):
    wid = _worker_id()
    lidx = lax.iota(jnp.int32, L)
    for g in range(2):
        pltpu.sync_copy(asT_h.at[pl.ds(g * N4P, N4P)], as_v)
        pltpu.sync_copy(adT_h.at[pl.ds(g * N4P, N4P)], ad_v)
        pltpu.sync_copy(c0_h.at[pl.ds(g * L, L)], c0_v)
        c0vec = c0_v[...]
        _zero_loop(den_v, N4P // L)

        def chunk(i, _, g=g):
            base = wid * EW + i * K1
            pltpu.sync_copy(src_h.at[pl.ds(base, K1)], sv)
            pltpu.sync_copy(dst_h.at[pl.ds(base, K1)], dv)

            def step(j, _):
                epos = j * 4 + (lidx >> 2)
                s4 = plsc.load_gather(sv, [epos])
                d4 = plsc.load_gather(dv, [epos])
                his = s4 * 4 + (lidx & 3)
                hdd = d4 * 4 + (lidx & 3)
                a = plsc.load_gather(as_v, [his])
                b = plsc.load_gather(ad_v, [hdd])
                t = a + b
                e = jnp.where(t > 0, t, t * jnp.float32(0.2))
                ex = jnp.exp(e - c0vec)
                plsc.addupdate_scatter(den_v, [hdd], ex)
                exv[pl.ds(j * L, L)] = ex
                return _

            lax.fori_loop(0, K1 // 4, step, None)
            pltpu.sync_copy(exv, ex_h.at[pl.ds(g * EE * 4 + base * 4, K1 * 4)])
            return _

        lax.fori_loop(0, EW // K1, chunk, None)
        pltpu.sync_copy(den_v, denp_h.at[pl.ds((g * NW + wid) * N4P, N4P)])


def _l1r_body(denp_h, den_h, acc_v, tmp_v):
    wid = _worker_id()
    for g in range(2):
        off = wid * SEG1
        pltpu.sync_copy(denp_h.at[pl.ds(g * NW * N4P + off, SEG1)], acc_v)
        for t in range(1, NW):
            pltpu.sync_copy(denp_h.at[pl.ds((g * NW + t) * N4P + off, SEG1)], tmp_v)
            _vadd_loop(acc_v, tmp_v, 0, SEG1 // L)
        pltpu.sync_copy(acc_v, den_h.at[pl.ds(g * N4P + off, SEG1)])


def _l1p2_body(g, src_h, dst_h, ex_h, den_h, hg_h,
               outp_h,
               den_v, sv, dv, exg_v, albuf, hbuf, obuf, sem, out_sh):
    cid = lax.axis_index("c")
    sid = lax.axis_index("s")
    wid = _worker_id()
    lidx = lax.iota(jnp.int32, L)

    pltpu.sync_copy(den_h.at[pl.ds(g * N4P, N4P)], den_v)
    _zero2d(obuf, K2, 32)
    _zero_out_shared(out_sh, obuf, sid)
    plsc.subcore_barrier()

    def chunk(i, _):
        base = wid * EW + i * K2
        pltpu.sync_copy(src_h.at[pl.ds(base, K2)], sv)
        pltpu.sync_copy(dst_h.at[pl.ds(base, K2)], dv)
        pltpu.async_copy(hg_h.at[sv], hbuf, sem).wait()
        pltpu.sync_copy(ex_h.at[pl.ds(g * EE * 4 + base * 4, K2 * 4)], exg_v)

        def astep(j, _):
            ex16 = exg_v[pl.ds(j * L, L)]
            d4 = plsc.load_gather(dv, [j * 4 + (lidx >> 2)])
            hdd = d4 * 4 + (lidx & 3)
            den16 = plsc.load_gather(den_v, [hdd])
            al = ex16 / (den16 + _EPS)
            aidx = (j * 4 + (lidx >> 2)) * 4 + (lidx & 3)
            plsc.store_scatter(albuf, [aidx], al)
            return _
        lax.fori_loop(0, K2 // 4, astep, None)

        def mstep(e, _):
            for q in range(2):
                av = plsc.load_gather(albuf, [e * 4 + q * 2 + (lidx >> 3)])
                obuf[e, pl.ds(q * L, L)] = av * hbuf[e, pl.ds(q * L, L)]
            return _
        lax.fori_loop(0, K2, mstep, None)

        pltpu.sync_copy(obuf, out_sh.at[dv], add=True)
        return _

    lax.fori_loop(0, EW // K2, chunk, None)
    plsc.subcore_barrier()
    _write_out_shared(out_sh, outp_h, cid, sid)


def _l2p1_body(src_h, dst_h, as_h, ad_h, c0_h,
               den_h, ex_h,
               as_v, ad_v, den_v, c0_v, sv, dv, exv, acc_v, tmp_v, den_sh):
    cid = lax.axis_index("c")
    sid = lax.axis_index("s")
    wid = _worker_id()
    pltpu.sync_copy(as_h, as_v)
    pltpu.sync_copy(ad_h, ad_v)
    pltpu.sync_copy(c0_h, c0_v)
    c0vec = c0_v[...]
    _zero_loop(den_v, NP // L)

    def chunk(i, _):
        base = wid * EW + i * K1
        pltpu.sync_copy(src_h.at[pl.ds(base, K1)], sv)
        pltpu.sync_copy(dst_h.at[pl.ds(base, K1)], dv)

        def step(j, _):
            s16 = sv[pl.ds(j * L, L)]
            d16 = dv[pl.ds(j * L, L)]
            a = plsc.load_gather(as_v, [s16])
            b = plsc.load_gather(ad_v, [d16])
            t = a + b
            e = jnp.where(t > 0, t, t * jnp.float32(0.2))
            ex = jnp.exp(e - c0vec)
            plsc.addupdate_scatter(den_v, [d16], ex)
            exv[pl.ds(j * L, L)] = ex
            return _

        lax.fori_loop(0, K1 // L, step, None)
        pltpu.sync_copy(exv, ex_h.at[pl.ds(base, K1)])
        return _

    lax.fori_loop(0, EW // K1, chunk, None)

    pltpu.sync_copy(den_v, den_sh.at[pl.ds(sid * NP, NP)])
    plsc.subcore_barrier()
    pltpu.sync_copy(den_sh.at[pl.ds(sid * SEG2, SEG2)], acc_v)
    for t in range(1, NS):
        pltpu.sync_copy(den_sh.at[pl.ds(t * NP + sid * SEG2, SEG2)], tmp_v)
        _vadd_loop(acc_v, tmp_v, 0, SEG2 // L)
    pltpu.sync_copy(acc_v, den_h.at[pl.ds(cid * NP + sid * SEG2, SEG2)])


def _l2p2_body(src_h, dst_h, ex_h, den_h, h2_h,
               outp_h,
               den_v, tmp_v, sv, dv, exv, albuf, hbuf, obuf, sem, out_sh):
    cid = lax.axis_index("c")
    sid = lax.axis_index("s")
    wid = _worker_id()
    lidx = lax.iota(jnp.int32, L)

    pltpu.sync_copy(den_h.at[pl.ds(0, NP)], den_v)
    for t in range(NS):
        pltpu.sync_copy(den_h.at[pl.ds(NP + t * SEG2, SEG2)], tmp_v)
        _vadd_loop(den_v, tmp_v, t * SEG2, SEG2 // L)

    _zero2d(obuf, K2, CP)
    _zero_out_shared(out_sh, obuf, sid)
    plsc.subcore_barrier()

    def chunk(i, _):
        base = wid * EW + i * K2
        pltpu.sync_copy(src_h.at[pl.ds(base, K2)], sv)
        pltpu.sync_copy(dst_h.at[pl.ds(base, K2)], dv)
        pltpu.async_copy(h2_h.at[sv], hbuf, sem).wait()
        pltpu.sync_copy(ex_h.at[pl.ds(base, K2)], exv)

        def astep(j, _):
            d16 = dv[pl.ds(j * L, L)]
            den16 = plsc.load_gather(den_v, [d16])
            al = exv[pl.ds(j * L, L)] / (den16 + _EPS)
            albuf[pl.ds(j * L, L)] = al
            return _
        lax.fori_loop(0, K2 // L, astep, None)

        def mstep(e, _):
            av = plsc.load_gather(albuf, [lidx * 0 + e])
            for q in range(CP // L):
                obuf[e, pl.ds(q * L, L)] = av * hbuf[e, pl.ds(q * L, L)]
            return _
        lax.fori_loop(0, K2, mstep, None)

        pltpu.sync_copy(obuf, out_sh.at[dv], add=True)
        return _

    lax.fori_loop(0, EW // K2, chunk, None)
    plsc.subcore_barrier()
    _write_out_shared(out_sh, outp_h, cid, sid)


def _sc_mesh():
    return plsc.VectorSubcoreMesh(
        core_axis_name="c", subcore_axis_name="s", num_cores=NC, num_subcores=NS)


_SC_PARAMS = pltpu.CompilerParams(
    needs_layout_passes=False, use_tc_tiling_on_sc=False)


def kernel(x, edge_index, W1, att_src1, att_dst1, b1, W2, att_src2, att_dst2, b2):
    f32 = jnp.float32
    src = edge_index[0]
    dst = edge_index[1]

    asf = att_src1.reshape(64)
    adf = att_dst1.reshape(64)
    hm = (jnp.arange(64)[:, None] // 8 == jnp.arange(8)[None, :]).astype(f32)
    asmask = hm * asf[:, None]
    admask = hm * adf[:, None]

    h1, a_s, a_d, c0 = pl.pallas_call(
        _tc_a_body,
        out_shape=(
            jax.ShapeDtypeStruct((NN, 64), f32),
            jax.ShapeDtypeStruct((NN, 8), f32),
            jax.ShapeDtypeStruct((NN, 8), f32),
            jax.ShapeDtypeStruct((1, 8), f32),
        ),
    )(x, W1, asmask, admask)

    def to_groups(a):
        g = a.reshape(NN, 2, 4).transpose(1, 0, 2).reshape(2, NN * 4)
        return jnp.pad(g, ((0, 0), (0, N4P - NN * 4))).reshape(2 * N4P)

    asT = to_groups(a_s)
    adT = to_groups(a_d)
    c0dup = jnp.tile(c0.reshape(2, 4), (1, 4)).reshape(2 * L)
    h1a = h1[:, :32]
    h1b = h1[:, 32:]

    mesh = _sc_mesh()

    denp1, ex1 = pl.kernel(
        _l1p1_body, mesh=mesh, compiler_params=_SC_PARAMS,
        out_type=(
            jax.ShapeDtypeStruct((2 * NW * N4P,), f32),
            jax.ShapeDtypeStruct((2 * EE * 4,), f32),
        ),
        scratch_types=[
            pltpu.VMEM((N4P,), f32),
            pltpu.VMEM((N4P,), f32),
            pltpu.VMEM((N4P,), f32),
            pltpu.VMEM((L,), f32),
            pltpu.VMEM((K1,), jnp.int32),
            pltpu.VMEM((K1,), jnp.int32),
            pltpu.VMEM((K1 * 4,), f32),
        ],
    )(src, dst, asT, adT, c0dup)

    den1 = pl.kernel(
        _l1r_body, mesh=mesh, compiler_params=_SC_PARAMS,
        out_type=jax.ShapeDtypeStruct((2 * N4P,), f32),
        scratch_types=[
            pltpu.VMEM((SEG1,), f32),
            pltpu.VMEM((SEG1,), f32),
        ],
    )(denp1)

    def l1p2(g, hg):
        return pl.kernel(
            functools.partial(_l1p2_body, g), mesh=mesh,
            compiler_params=_SC_PARAMS,
            out_type=jax.ShapeDtypeStruct((NC, NN, 32), f32),
            scratch_types=[
                pltpu.VMEM((N4P,), f32),
                pltpu.VMEM((K2,), jnp.int32),
                pltpu.VMEM((K2,), jnp.int32),
                pltpu.VMEM((K2 * 4,), f32),
                pltpu.VMEM((K2 * 4,), f32),
                pltpu.VMEM((K2, 32), f32),
                pltpu.VMEM((K2, 32), f32),
                pltpu.SemaphoreType.DMA,
                pltpu.VMEM_SHARED((NN, 32), f32),
            ],
        )(src, dst, ex1, den1, hg)

    outpa = l1p2(0, h1a)
    outpb = l1p2(1, h1b)

    w2p = jnp.pad(W2, ((0, 0), (0, CP - 40)))
    a2sp = jnp.pad(att_src2.reshape(40, 1), ((0, CP - 40), (0, 0)))
    a2dp = jnp.pad(att_dst2.reshape(40, 1), ((0, CP - 40), (0, 0)))
    emb, h2, as2, ad2, c02 = pl.pallas_call(
        _tc_b_body,
        out_shape=(
            jax.ShapeDtypeStruct((NN, 64), f32),
            jax.ShapeDtypeStruct((NN, CP), f32),
            jax.ShapeDtypeStruct((NN, 1), f32),
            jax.ShapeDtypeStruct((NN, 1), f32),
            jax.ShapeDtypeStruct((1, 1), f32),
        ),
    )(outpa, outpb, b1.reshape(1, 64), w2p, a2sp, a2dp)

    as2p = jnp.pad(as2.reshape(NN), (0, NP - NN))
    ad2p = jnp.pad(ad2.reshape(NN), (0, NP - NN))
    c02dup = jnp.broadcast_to(c02.reshape(1), (L,))

    den2, ex2 = pl.kernel(
        _l2p1_body, mesh=mesh, compiler_params=_SC_PARAMS,
        out_type=(
            jax.ShapeDtypeStruct((NC * NP,), f32),
            jax.ShapeDtypeStruct((EE,), f32),
        ),
        scratch_types=[
            pltpu.VMEM((NP,), f32),
            pltpu.VMEM((NP,), f32),
            pltpu.VMEM((NP,), f32),
            pltpu.VMEM((L,), f32),
            pltpu.VMEM((K1,), jnp.int32),
            pltpu.VMEM((K1,), jnp.int32),
            pltpu.VMEM((K1,), f32),
            pltpu.VMEM((SEG2,), f32),
            pltpu.VMEM((SEG2,), f32),
            pltpu.VMEM_SHARED((NS * NP,), f32),
        ],
    )(src, dst, as2p, ad2p, c02dup)

    outp2 = pl.kernel(
        _l2p2_body, mesh=mesh, compiler_params=_SC_PARAMS,
        out_type=jax.ShapeDtypeStruct((NC, NN, CP), f32),
        scratch_types=[
            pltpu.VMEM((NP,), f32),
            pltpu.VMEM((SEG2,), f32),
            pltpu.VMEM((K2,), jnp.int32),
            pltpu.VMEM((K2,), jnp.int32),
            pltpu.VMEM((K2,), f32),
            pltpu.VMEM((K2,), f32),
            pltpu.VMEM((K2, CP), f32),
            pltpu.VMEM((K2, CP), f32),
            pltpu.SemaphoreType.DMA,
            pltpu.VMEM_SHARED((NN, CP), f32),
        ],
    )(src, dst, ex2, den2, h2)

    b2p = jnp.pad(b2, (0, CP - 40)).reshape(1, CP)
    out48 = pl.pallas_call(
        _tc_c_body,
        out_shape=jax.ShapeDtypeStruct((NN, CP), f32),
    )(outp2, b2p)

    return (out48[:, :40], emb)


# speedup vs baseline: 41.7040x; 1.5125x over previous
import functools

import jax
import jax.numpy as jnp
from jax import lax
from jax.experimental import pallas as pl
from jax.experimental.pallas import tpu as pltpu
from jax.experimental.pallas import tpu_sc as plsc

NN = 10000
EE = 320000
NC = 2
NS = 16
NW = NC * NS
EW = EE // NW
L = 16

N4P = 40960
SEG1 = N4P // NW
NP = 10240
SEG2 = NP // NS
K1 = 400
K2 = 80
CP = 48
RT = 624
RTB = NS * RT
REX = NN - RTB

_EPS = 1e-16


def _tc_a_body(x_ref, w1_ref, asm_ref, adm_ref, h1_ref, as_ref, ad_ref, c0_ref):
    h1 = jnp.dot(x_ref[...], w1_ref[...], preferred_element_type=jnp.float32)
    h1_ref[...] = h1
    a_s = jnp.dot(h1, asm_ref[...], preferred_element_type=jnp.float32)
    a_d = jnp.dot(h1, adm_ref[...], preferred_element_type=jnp.float32)
    as_ref[...] = a_s
    ad_ref[...] = a_d
    c0 = jnp.max(a_s, axis=0, keepdims=True) + jnp.max(a_d, axis=0, keepdims=True)
    c0_ref[...] = jnp.maximum(c0, 0.0)


def _tc_b_body(pa_ref, pb_ref, b1_ref, w2_ref, a2s_ref, a2d_ref,
               emb_ref, h2_ref, as2_ref, ad2_ref, c02_ref):
    ha = pa_ref[0] + pa_ref[1]
    hb = pb_ref[0] + pb_ref[1]
    h1 = jnp.concatenate([ha, hb], axis=1) + b1_ref[...]
    emb = jnp.where(h1 > 0, h1, jnp.exp(jnp.minimum(h1, 0.0)) - 1.0)
    emb_ref[...] = emb
    h2 = jnp.dot(emb, w2_ref[...], preferred_element_type=jnp.float32)
    h2_ref[...] = h2
    as2 = jnp.dot(h2, a2s_ref[...], preferred_element_type=jnp.float32)
    ad2 = jnp.dot(h2, a2d_ref[...], preferred_element_type=jnp.float32)
    as2_ref[...] = as2
    ad2_ref[...] = ad2
    c02 = jnp.max(as2, axis=0, keepdims=True) + jnp.max(ad2, axis=0, keepdims=True)
    c02_ref[...] = jnp.maximum(c02, 0.0)


def _tc_c_body(p_ref, b2_ref, out_ref):
    o = p_ref[0] + p_ref[1] + b2_ref[...]
    mask = lax.broadcasted_iota(jnp.int32, (NN, CP), 1) < 40
    xm = jnp.where(mask, o, jnp.float32(-1e30))
    m = jnp.max(xm, axis=1, keepdims=True)
    ex = jnp.where(mask, jnp.exp(o - m), 0.0)
    lse = jnp.log(jnp.sum(ex, axis=1, keepdims=True))
    out_ref[...] = o - m - lse


def _worker_id():
    return lax.axis_index("s") * NC + lax.axis_index("c")


def _vadd_loop(dst_ref, src_ref, dst_off, n_vregs):
    def body(i, _):
        o = dst_off + i * L
        dst_ref[pl.ds(o, L)] = dst_ref[pl.ds(o, L)] + src_ref[pl.ds(i * L, L)]
        return _
    lax.fori_loop(0, n_vregs, body, None)


def _zero_loop(dst_ref, n_vregs, dtype=jnp.float32):
    z = jnp.zeros((L,), dtype)
    def body(i, _):
        dst_ref[pl.ds(i * L, L)] = z
        return _
    lax.fori_loop(0, n_vregs, body, None)


def _zero2d(dst_ref, nrows, ncols):
    z = jnp.zeros((L,), jnp.float32)
    def body(r, _):
        for q in range(ncols // L):
            dst_ref[r, pl.ds(q * L, L)] = z
        return _
    lax.fori_loop(0, nrows, body, None)


def _zero_out_shared(out_sh, obuf, sid):
    stripe = sid * RT
    for r in range(0, RT - K2 + 1, K2):
        pltpu.sync_copy(obuf, out_sh.at[pl.ds(stripe + r, K2)])
    pltpu.sync_copy(obuf.at[pl.ds(0, RT - 7 * K2)],
                    out_sh.at[pl.ds(stripe + 7 * K2, RT - 7 * K2)])

    @pl.when(sid == 0)
    def _():
        pltpu.sync_copy(obuf.at[pl.ds(0, REX)], out_sh.at[pl.ds(RTB, REX)])


def _write_out_shared(out_sh, outp_h, cid, sid):
    stripe = sid * RT
    pltpu.sync_copy(out_sh.at[pl.ds(stripe, RT)],
                    outp_h.at[cid, pl.ds(stripe, RT)])

    @pl.when(sid == 0)
    def _():
        pltpu.sync_copy(out_sh.at[pl.ds(RTB, REX)],
                        outp_h.at[cid, pl.ds(RTB, REX)])


def _l1p1_body(src_h, dst_h, asT_h, adT_h, c0_h,
               denp_h, ex_h,
               as_v, ad_v, den_v, c0_v, sv, dv, exv):
    wid = _worker_id()
    lidx = lax.iota(jnp.int32, L)
    for g in range(2):
        pltpu.sync_copy(asT_h.at[pl.ds(g * N4P, N4P)], as_v)
        pltpu.sync_copy(adT_h.at[pl.ds(g * N4P, N4P)], ad_v)
        pltpu.sync_copy(c0_h.at[pl.ds(g * L, L)], c0_v)
        c0vec = c0_v[...]
        _zero_loop(den_v, N4P // L)

        def chunk(i, _, g=g):
            base = wid * EW + i * K1
            pltpu.sync_copy(src_h.at[pl.ds(base, K1)], sv)
            pltpu.sync_copy(dst_h.at[pl.ds(base, K1)], dv)

            def step(j, _):
                epos = j * 4 + (lidx >> 2)
                s4 = plsc.load_gather(sv, [epos])
                d4 = plsc.load_gather(dv, [epos])
                his = s4 * 4 + (lidx & 3)
                hdd = d4 * 4 + (lidx & 3)
                a = plsc.load_gather(as_v, [his])
                b = plsc.load_gather(ad_v, [hdd])
                t = a + b
                e = jnp.where(t > 0, t, t * jnp.float32(0.2))
                ex = jnp.exp(e - c0vec)
                plsc.addupdate_scatter(den_v, [hdd], ex)
                exv[pl.ds(j * L, L)] = ex
                return _

            lax.fori_loop(0, K1 // 4, step, None)
            pltpu.sync_copy(exv, ex_h.at[pl.ds(g * EE * 4 + base * 4, K1 * 4)])
            return _

        lax.fori_loop(0, EW // K1, chunk, None)
        pltpu.sync_copy(den_v, denp_h.at[pl.ds((g * NW + wid) * N4P, N4P)])


def _l1r_body(denp_h, den_h, acc_v, tmp_v):
    wid = _worker_id()
    for g in range(2):
        off = wid * SEG1
        pltpu.sync_copy(denp_h.at[pl.ds(g * NW * N4P + off, SEG1)], acc_v)
        for t in range(1, NW):
            pltpu.sync_copy(denp_h.at[pl.ds((g * NW + t) * N4P + off, SEG1)], tmp_v)
            _vadd_loop(acc_v, tmp_v, 0, SEG1 // L)
        pltpu.sync_copy(acc_v, den_h.at[pl.ds(g * N4P + off, SEG1)])


def _copy_idx(dst_ref, src_ref):
    for k in range(K2 // L):
        dst_ref[pl.ds(k * L, L)] = src_ref[pl.ds(k * L, L)]


def _agg_pipeline(src_h, dst_h, wid, svs, dvs, dvsc, hbufs, obufs, out_sh,
                  sems, issue_ex, wait_ex, issue_h, wait_h, compute):
    ncheck = EW // K2
    assert ncheck % 2 == 1
    sem_sv, sem_dv, sem_s = sems

    def issue_prefetch(c, p):
        base = wid * EW + c * K2
        pltpu.async_copy(src_h.at[pl.ds(base, K2)], svs[p], sem_sv[p])
        pltpu.async_copy(dst_h.at[pl.ds(base, K2)], dvs[p], sem_dv[p])
        issue_ex(c, p)

    def wait_prefetch(p):
        pltpu.make_async_copy(src_h.at[pl.ds(0, K2)], svs[p], sem_sv[p]).wait()
        pltpu.make_async_copy(dst_h.at[pl.ds(0, K2)], dvs[p], sem_dv[p]).wait()
        wait_ex(p)

    def issue_scatter(p):
        pltpu.async_copy(obufs[p], out_sh.at[dvsc[p]], sem_s[p], add=True)

    def wait_scatter(p):
        pltpu.make_async_copy(obufs[p], out_sh.at[dvsc[p]], sem_s[p]).wait()

    issue_scatter(0)
    issue_scatter(1)
    issue_prefetch(0, 0)
    wait_prefetch(0)
    issue_h(0)
    issue_prefetch(1, 1)

    def diter(it, _):
        for p in range(2):
            i = it * 2 + p
            q = 1 - p
            wait_scatter(p)
            wait_prefetch(q)
            issue_h(q)
            wait_h(p)
            compute(p)
            _copy_idx(dvsc[p], dvs[p])
            issue_scatter(p)
            issue_prefetch(jnp.minimum(i + 2, ncheck - 1), p)
        return _

    lax.fori_loop(0, (ncheck - 1) // 2, diter, None)
    wait_scatter(0)
    wait_prefetch(1)
    wait_h(0)
    compute(0)
    _copy_idx(dvsc[0], dvs[0])
    issue_scatter(0)
    wait_scatter(0)
    wait_scatter(1)


def _l1p2_body(g, src_h, dst_h, ex_h, den_h, hg_h,
               outp_h,
               den_v, albuf,
               sv0, sv1, dv0, dv1, dvc0, dvc1, ex0, ex1,
               hb0, hb1, ob0, ob1,
               ssv0, ssv1, sdv0, sdv1, sex0, sex1, sh0, sh1, ss0, ss1,
               out_sh):
    cid = lax.axis_index("c")
    sid = lax.axis_index("s")
    wid = _worker_id()
    lidx = lax.iota(jnp.int32, L)
    svs, dvs, dvsc = (sv0, sv1), (dv0, dv1), (dvc0, dvc1)
    exs, hbufs, obufs = (ex0, ex1), (hb0, hb1), (ob0, ob1)
    sem_sv, sem_dv, sem_ex = (ssv0, ssv1), (sdv0, sdv1), (sex0, sex1)
    sem_h, sem_s = (sh0, sh1), (ss0, ss1)

    pltpu.sync_copy(den_h.at[pl.ds(g * N4P, N4P)], den_v)
    _zero2d(ob0, K2, 32)
    _zero2d(ob1, K2, 32)
    _zero_loop(dvc0, K2 // L, jnp.int32)
    _zero_loop(dvc1, K2 // L, jnp.int32)
    _zero_out_shared(out_sh, ob0, sid)
    plsc.subcore_barrier()

    def issue_ex(c, p):
        base = wid * EW + c * K2
        pltpu.async_copy(ex_h.at[pl.ds(g * EE * 4 + base * 4, K2 * 4)],
                         exs[p], sem_ex[p])

    def wait_ex(p):
        pltpu.make_async_copy(ex_h.at[pl.ds(0, K2 * 4)], exs[p],
                              sem_ex[p]).wait()

    def issue_h(p):
        pltpu.async_copy(hg_h.at[svs[p]], hbufs[p], sem_h[p])

    def wait_h(p):
        pltpu.make_async_copy(hg_h.at[svs[p]], hbufs[p], sem_h[p]).wait()

    def compute(p):
        exg_v, dv, hbuf, obuf = exs[p], dvs[p], hbufs[p], obufs[p]

        def astep(j, _):
            ex16 = exg_v[pl.ds(j * L, L)]
            d4 = plsc.load_gather(dv, [j * 4 + (lidx >> 2)])
            hdd = d4 * 4 + (lidx & 3)
            den16 = plsc.load_gather(den_v, [hdd])
            al = ex16 / (den16 + _EPS)
            aidx = (j * 4 + (lidx >> 2)) * 4 + (lidx & 3)
            plsc.store_scatter(albuf, [aidx], al)
            return _
        lax.fori_loop(0, K2 // 4, astep, None)

        def mstep(e, _):
            for q in range(2):
                av = plsc.load_gather(albuf, [e * 4 + q * 2 + (lidx >> 3)])
                obuf[e, pl.ds(q * L, L)] = av * hbuf[e, pl.ds(q * L, L)]
            return _
        lax.fori_loop(0, K2, mstep, None)

    _agg_pipeline(src_h, dst_h, wid, svs, dvs, dvsc, hbufs, obufs, out_sh,
                  (sem_sv, sem_dv, sem_s), issue_ex, wait_ex,
                  issue_h, wait_h, compute)
    plsc.subcore_barrier()
    _write_out_shared(out_sh, outp_h, cid, sid)


def _l2p1_body(src_h, dst_h, as_h, ad_h, c0_h,
               den_h, ex_h,
               as_v, ad_v, den_v, c0_v, sv, dv, exv, acc_v, tmp_v, den_sh):
    cid = lax.axis_index("c")
    sid = lax.axis_index("s")
    wid = _worker_id()
    pltpu.sync_copy(as_h, as_v)
    pltpu.sync_copy(ad_h, ad_v)
    pltpu.sync_copy(c0_h, c0_v)
    c0vec = c0_v[...]
    _zero_loop(den_v, NP // L)

    def chunk(i, _):
        base = wid * EW + i * K1
        pltpu.sync_copy(src_h.at[pl.ds(base, K1)], sv)
        pltpu.sync_copy(dst_h.at[pl.ds(base, K1)], dv)

        def step(j, _):
            s16 = sv[pl.ds(j * L, L)]
            d16 = dv[pl.ds(j * L, L)]
            a = plsc.load_gather(as_v, [s16])
            b = plsc.load_gather(ad_v, [d16])
            t = a + b
            e = jnp.where(t > 0, t, t * jnp.float32(0.2))
            ex = jnp.exp(e - c0vec)
            plsc.addupdate_scatter(den_v, [d16], ex)
            exv[pl.ds(j * L, L)] = ex
            return _

        lax.fori_loop(0, K1 // L, step, None)
        pltpu.sync_copy(exv, ex_h.at[pl.ds(base, K1)])
        return _

    lax.fori_loop(0, EW // K1, chunk, None)

    pltpu.sync_copy(den_v, den_sh.at[pl.ds(sid * NP, NP)])
    plsc.subcore_barrier()
    pltpu.sync_copy(den_sh.at[pl.ds(sid * SEG2, SEG2)], acc_v)
    for t in range(1, NS):
        pltpu.sync_copy(den_sh.at[pl.ds(t * NP + sid * SEG2, SEG2)], tmp_v)
        _vadd_loop(acc_v, tmp_v, 0, SEG2 // L)
    pltpu.sync_copy(acc_v, den_h.at[pl.ds(cid * NP + sid * SEG2, SEG2)])


def _l2p2_body(src_h, dst_h, ex_h, den_h, h2_h,
               outp_h,
               den_v, tmp_v, albuf,
               sv0, sv1, dv0, dv1, dvc0, dvc1, ex0, ex1,
               hb0, hb1, ob0, ob1,
               ssv0, ssv1, sdv0, sdv1, sex0, sex1, sh0, sh1, ss0, ss1,
               out_sh):
    cid = lax.axis_index("c")
    sid = lax.axis_index("s")
    wid = _worker_id()
    lidx = lax.iota(jnp.int32, L)
    svs, dvs, dvsc = (sv0, sv1), (dv0, dv1), (dvc0, dvc1)
    exs, hbufs, obufs = (ex0, ex1), (hb0, hb1), (ob0, ob1)
    sem_sv, sem_dv, sem_ex = (ssv0, ssv1), (sdv0, sdv1), (sex0, sex1)
    sem_h, sem_s = (sh0, sh1), (ss0, ss1)

    pltpu.sync_copy(den_h.at[pl.ds(0, NP)], den_v)
    for t in range(NS):
        pltpu.sync_copy(den_h.at[pl.ds(NP + t * SEG2, SEG2)], tmp_v)
        _vadd_loop(den_v, tmp_v, t * SEG2, SEG2 // L)

    _zero2d(ob0, K2, CP)
    _zero2d(ob1, K2, CP)
    _zero_loop(dvc0, K2 // L, jnp.int32)
    _zero_loop(dvc1, K2 // L, jnp.int32)
    _zero_out_shared(out_sh, ob0, sid)
    plsc.subcore_barrier()

    def issue_ex(c, p):
        base = wid * EW + c * K2
        pltpu.async_copy(ex_h.at[pl.ds(base, K2)], exs[p], sem_ex[p])

    def wait_ex(p):
        pltpu.make_async_copy(ex_h.at[pl.ds(0, K2)], exs[p], sem_ex[p]).wait()

    def issue_h(p):
        pltpu.async_copy(h2_h.at[svs[p]], hbufs[p], sem_h[p])

    def wait_h(p):
        pltpu.make_async_copy(h2_h.at[svs[p]], hbufs[p], sem_h[p]).wait()

    def compute(p):
        exv, dv, hbuf, obuf = exs[p], dvs[p], hbufs[p], obufs[p]

        def astep(j, _):
            d16 = dv[pl.ds(j * L, L)]
            den16 = plsc.load_gather(den_v, [d16])
            al = exv[pl.ds(j * L, L)] / (den16 + _EPS)
            albuf[pl.ds(j * L, L)] = al
            return _
        lax.fori_loop(0, K2 // L, astep, None)

        def mstep(e, _):
            av = plsc.load_gather(albuf, [lidx * 0 + e])
            for q in range(CP // L):
                obuf[e, pl.ds(q * L, L)] = av * hbuf[e, pl.ds(q * L, L)]
            return _
        lax.fori_loop(0, K2, mstep, None)

    _agg_pipeline(src_h, dst_h, wid, svs, dvs, dvsc, hbufs, obufs, out_sh,
                  (sem_sv, sem_dv, sem_s), issue_ex, wait_ex,
                  issue_h, wait_h, compute)
    plsc.subcore_barrier()
    _write_out_shared(out_sh, outp_h, cid, sid)


def _sc_mesh():
    return plsc.VectorSubcoreMesh(
        core_axis_name="c", subcore_axis_name="s", num_cores=NC, num_subcores=NS)


_SC_PARAMS = pltpu.CompilerParams(
    needs_layout_passes=False, use_tc_tiling_on_sc=False)


def kernel(x, edge_index, W1, att_src1, att_dst1, b1, W2, att_src2, att_dst2, b2):
    f32 = jnp.float32
    src = edge_index[0]
    dst = edge_index[1]

    asf = att_src1.reshape(64)
    adf = att_dst1.reshape(64)
    hm = (jnp.arange(64)[:, None] // 8 == jnp.arange(8)[None, :]).astype(f32)
    asmask = hm * asf[:, None]
    admask = hm * adf[:, None]

    h1, a_s, a_d, c0 = pl.pallas_call(
        _tc_a_body,
        out_shape=(
            jax.ShapeDtypeStruct((NN, 64), f32),
            jax.ShapeDtypeStruct((NN, 8), f32),
            jax.ShapeDtypeStruct((NN, 8), f32),
            jax.ShapeDtypeStruct((1, 8), f32),
        ),
    )(x, W1, asmask, admask)

    def to_groups(a):
        g = a.reshape(NN, 2, 4).transpose(1, 0, 2).reshape(2, NN * 4)
        return jnp.pad(g, ((0, 0), (0, N4P - NN * 4))).reshape(2 * N4P)

    asT = to_groups(a_s)
    adT = to_groups(a_d)
    c0dup = jnp.tile(c0.reshape(2, 4), (1, 4)).reshape(2 * L)
    h1a = h1[:, :32]
    h1b = h1[:, 32:]

    mesh = _sc_mesh()

    denp1, ex1 = pl.kernel(
        _l1p1_body, mesh=mesh, compiler_params=_SC_PARAMS,
        out_type=(
            jax.ShapeDtypeStruct((2 * NW * N4P,), f32),
            jax.ShapeDtypeStruct((2 * EE * 4,), f32),
        ),
        scratch_types=[
            pltpu.VMEM((N4P,), f32),
            pltpu.VMEM((N4P,), f32),
            pltpu.VMEM((N4P,), f32),
            pltpu.VMEM((L,), f32),
            pltpu.VMEM((K1,), jnp.int32),
            pltpu.VMEM((K1,), jnp.int32),
            pltpu.VMEM((K1 * 4,), f32),
        ],
    )(src, dst, asT, adT, c0dup)

    den1 = pl.kernel(
        _l1r_body, mesh=mesh, compiler_params=_SC_PARAMS,
        out_type=jax.ShapeDtypeStruct((2 * N4P,), f32),
        scratch_types=[
            pltpu.VMEM((SEG1,), f32),
            pltpu.VMEM((SEG1,), f32),
        ],
    )(denp1)

    def l1p2(g, hg):
        return pl.kernel(
            functools.partial(_l1p2_body, g), mesh=mesh,
            compiler_params=_SC_PARAMS,
            out_type=jax.ShapeDtypeStruct((NC, NN, 32), f32),
            scratch_types=(
                [pltpu.VMEM((N4P,), f32), pltpu.VMEM((K2 * 4,), f32)]
                + [pltpu.VMEM((K2,), jnp.int32)] * 6
                + [pltpu.VMEM((K2 * 4,), f32)] * 2
                + [pltpu.VMEM((K2, 32), f32)] * 4
                + [pltpu.SemaphoreType.DMA] * 10
                + [pltpu.VMEM_SHARED((NN, 32), f32)]
            ),
        )(src, dst, ex1, den1, hg)

    outpa = l1p2(0, h1a)
    outpb = l1p2(1, h1b)

    w2p = jnp.pad(W2, ((0, 0), (0, CP - 40)))
    a2sp = jnp.pad(att_src2.reshape(40, 1), ((0, CP - 40), (0, 0)))
    a2dp = jnp.pad(att_dst2.reshape(40, 1), ((0, CP - 40), (0, 0)))
    emb, h2, as2, ad2, c02 = pl.pallas_call(
        _tc_b_body,
        out_shape=(
            jax.ShapeDtypeStruct((NN, 64), f32),
            jax.ShapeDtypeStruct((NN, CP), f32),
            jax.ShapeDtypeStruct((NN, 1), f32),
            jax.ShapeDtypeStruct((NN, 1), f32),
            jax.ShapeDtypeStruct((1, 1), f32),
        ),
    )(outpa, outpb, b1.reshape(1, 64), w2p, a2sp, a2dp)

    as2p = jnp.pad(as2.reshape(NN), (0, NP - NN))
    ad2p = jnp.pad(ad2.reshape(NN), (0, NP - NN))
    c02dup = jnp.broadcast_to(c02.reshape(1), (L,))

    den2, ex2 = pl.kernel(
        _l2p1_body, mesh=mesh, compiler_params=_SC_PARAMS,
        out_type=(
            jax.ShapeDtypeStruct((NC * NP,), f32),
            jax.ShapeDtypeStruct((EE,), f32),
        ),
        scratch_types=[
            pltpu.VMEM((NP,), f32),
            pltpu.VMEM((NP,), f32),
            pltpu.VMEM((NP,), f32),
            pltpu.VMEM((L,), f32),
            pltpu.VMEM((K1,), jnp.int32),
            pltpu.VMEM((K1,), jnp.int32),
            pltpu.VMEM((K1,), f32),
            pltpu.VMEM((SEG2,), f32),
            pltpu.VMEM((SEG2,), f32),
            pltpu.VMEM_SHARED((NS * NP,), f32),
        ],
    )(src, dst, as2p, ad2p, c02dup)

    outp2 = pl.kernel(
        _l2p2_body, mesh=mesh, compiler_params=_SC_PARAMS,
        out_type=jax.ShapeDtypeStruct((NC, NN, CP), f32),
        scratch_types=(
            [pltpu.VMEM((NP,), f32), pltpu.VMEM((SEG2,), f32),
             pltpu.VMEM((K2,), f32)]
            + [pltpu.VMEM((K2,), jnp.int32)] * 6
            + [pltpu.VMEM((K2,), f32)] * 2
            + [pltpu.VMEM((K2, CP), f32)] * 4
            + [pltpu.SemaphoreType.DMA] * 10
            + [pltpu.VMEM_SHARED((NN, CP), f32)]
        ),
    )(src, dst, ex2, den2, h2)

    b2p = jnp.pad(b2, (0, CP - 40)).reshape(1, CP)
    out48 = pl.pallas_call(
        _tc_c_body,
        out_shape=jax.ShapeDtypeStruct((NN, CP), f32),
    )(outp2, b2p)

    return (out48[:, :40], emb)


# speedup vs baseline: 43.5958x; 1.0454x over previous
import functools

import jax
import jax.numpy as jnp
from jax import lax
from jax.experimental import pallas as pl
from jax.experimental.pallas import tpu as pltpu
from jax.experimental.pallas import tpu_sc as plsc

NN = 10000
EE = 320000
NC = 2
NS = 16
NW = NC * NS
EW = EE // NW
L = 16

N4P = 40960
SEG1 = N4P // NW
NP = 10240
SEG2 = NP // NS
K1 = 400
K2 = 80
CP = 48
RT = 624
RTB = NS * RT
REX = NN - RTB

_EPS = 1e-16


def _tc_a_body(x_ref, w1_ref, asm_ref, adm_ref, h1_ref, as_ref, ad_ref, c0_ref):
    h1 = jnp.dot(x_ref[...], w1_ref[...], preferred_element_type=jnp.float32)
    h1_ref[...] = h1
    a_s = jnp.dot(h1, asm_ref[...], preferred_element_type=jnp.float32)
    a_d = jnp.dot(h1, adm_ref[...], preferred_element_type=jnp.float32)
    as_ref[...] = a_s
    ad_ref[...] = a_d
    c0 = jnp.max(a_s, axis=0, keepdims=True) + jnp.max(a_d, axis=0, keepdims=True)
    c0_ref[...] = jnp.maximum(c0, 0.0)


def _tc_b_body(pa_ref, pb_ref, b1_ref, w2_ref, a2s_ref, a2d_ref,
               emb_ref, h2_ref, as2_ref, ad2_ref, c02_ref):
    ha = pa_ref[0] + pa_ref[1]
    hb = pb_ref[0] + pb_ref[1]
    h1 = jnp.concatenate([ha, hb], axis=1) + b1_ref[...]
    emb = jnp.where(h1 > 0, h1, jnp.exp(jnp.minimum(h1, 0.0)) - 1.0)
    emb_ref[...] = emb
    h2 = jnp.dot(emb, w2_ref[...], preferred_element_type=jnp.float32)
    h2_ref[...] = h2
    as2 = jnp.dot(h2, a2s_ref[...], preferred_element_type=jnp.float32)
    ad2 = jnp.dot(h2, a2d_ref[...], preferred_element_type=jnp.float32)
    as2_ref[...] = as2
    ad2_ref[...] = ad2
    c02 = jnp.max(as2, axis=0, keepdims=True) + jnp.max(ad2, axis=0, keepdims=True)
    c02_ref[...] = jnp.maximum(c02, 0.0)


def _tc_c_body(p_ref, b2_ref, out_ref):
    o = p_ref[0] + p_ref[1] + b2_ref[...]
    mask = lax.broadcasted_iota(jnp.int32, (NN, CP), 1) < 40
    xm = jnp.where(mask, o, jnp.float32(-1e30))
    m = jnp.max(xm, axis=1, keepdims=True)
    ex = jnp.where(mask, jnp.exp(o - m), 0.0)
    lse = jnp.log(jnp.sum(ex, axis=1, keepdims=True))
    out_ref[...] = o - m - lse


def _worker_id():
    return lax.axis_index("s") * NC + lax.axis_index("c")


def _vadd_loop(dst_ref, src_ref, dst_off, n_vregs):
    def body(i, _):
        o = dst_off + i * L
        dst_ref[pl.ds(o, L)] = dst_ref[pl.ds(o, L)] + src_ref[pl.ds(i * L, L)]
        return _
    lax.fori_loop(0, n_vregs, body, None)


def _zero_loop(dst_ref, n_vregs, dtype=jnp.float32):
    z = jnp.zeros((L,), dtype)
    def body(i, _):
        dst_ref[pl.ds(i * L, L)] = z
        return _
    lax.fori_loop(0, n_vregs, body, None)


def _zero2d(dst_ref, nrows, ncols):
    z = jnp.zeros((L,), jnp.float32)
    def body(r, _):
        for q in range(ncols // L):
            dst_ref[r, pl.ds(q * L, L)] = z
        return _
    lax.fori_loop(0, nrows, body, None)


def _zero_out_shared(out_sh, obuf, sid):
    stripe = sid * RT
    for r in range(0, RT - K2 + 1, K2):
        pltpu.sync_copy(obuf, out_sh.at[pl.ds(stripe + r, K2)])
    pltpu.sync_copy(obuf.at[pl.ds(0, RT - 7 * K2)],
                    out_sh.at[pl.ds(stripe + 7 * K2, RT - 7 * K2)])

    @pl.when(sid == 0)
    def _():
        pltpu.sync_copy(obuf.at[pl.ds(0, REX)], out_sh.at[pl.ds(RTB, REX)])


def _write_out_shared(out_sh, outp_h, cid, sid):
    stripe = sid * RT
    pltpu.sync_copy(out_sh.at[pl.ds(stripe, RT)],
                    outp_h.at[cid, pl.ds(stripe, RT)])

    @pl.when(sid == 0)
    def _():
        pltpu.sync_copy(out_sh.at[pl.ds(RTB, REX)],
                        outp_h.at[cid, pl.ds(RTB, REX)])


def _l1p1_body(src_h, dst_h, asT_h, adT_h, c0_h,
               denp_h, ex_h,
               as_v, ad_v, den_v, c0_v,
               sv0, sv1, dv0, dv1, exv0, exv1,
               ssv0, ssv1, sdv0, sdv1, sex0, sex1):
    wid = _worker_id()
    lidx = lax.iota(jnp.int32, L)
    svs, dvs, exvs = (sv0, sv1), (dv0, dv1), (exv0, exv1)
    sem_sv, sem_dv, sem_ex = (ssv0, ssv1), (sdv0, sdv1), (sex0, sex1)
    nch = EW // K1
    assert nch % 2 == 1

    def issue_prefetch(c, p):
        base = wid * EW + c * K1
        pltpu.async_copy(src_h.at[pl.ds(base, K1)], svs[p], sem_sv[p])
        pltpu.async_copy(dst_h.at[pl.ds(base, K1)], dvs[p], sem_dv[p])

    def wait_prefetch(p):
        pltpu.make_async_copy(src_h.at[pl.ds(0, K1)], svs[p], sem_sv[p]).wait()
        pltpu.make_async_copy(dst_h.at[pl.ds(0, K1)], dvs[p], sem_dv[p]).wait()

    def wait_ex(p):
        pltpu.make_async_copy(exv0, ex_h.at[pl.ds(0, K1 * 4)],
                              sem_ex[p]).wait()

    for g in range(2):
        pltpu.sync_copy(asT_h.at[pl.ds(g * N4P, N4P)], as_v)
        pltpu.sync_copy(adT_h.at[pl.ds(g * N4P, N4P)], ad_v)
        pltpu.sync_copy(c0_h.at[pl.ds(g * L, L)], c0_v)
        c0vec = c0_v[...]
        _zero_loop(den_v, N4P // L)

        def body(i, p, first, g=None, c0vec=None):
            sv, dv, exv = svs[p], dvs[p], exvs[p]
            if not first:
                wait_ex(p)
            wait_prefetch(p)

            def step(j, _):
                epos = j * 4 + (lidx >> 2)
                s4 = plsc.load_gather(sv, [epos])
                d4 = plsc.load_gather(dv, [epos])
                his = s4 * 4 + (lidx & 3)
                hdd = d4 * 4 + (lidx & 3)
                a = plsc.load_gather(as_v, [his])
                b = plsc.load_gather(ad_v, [hdd])
                t = a + b
                e = jnp.where(t > 0, t, t * jnp.float32(0.2))
                ex = jnp.exp(e - c0vec)
                plsc.addupdate_scatter(den_v, [hdd], ex)
                exv[pl.ds(j * L, L)] = ex
                return _

            lax.fori_loop(0, K1 // 4, step, None)
            base = wid * EW + i * K1
            pltpu.async_copy(exv, ex_h.at[pl.ds(g * EE * 4 + base * 4, K1 * 4)],
                             sem_ex[p])
            return base

        issue_prefetch(0, 0)
        issue_prefetch(1, 1)
        body(0, 0, True, g=g, c0vec=c0vec)
        issue_prefetch(2, 0)
        body(1, 1, True, g=g, c0vec=c0vec)
        issue_prefetch(3, 1)

        def diter(it, _, g=g, c0vec=c0vec):
            for p in range(2):
                i = it * 2 + p
                body(i, p, False, g=g, c0vec=c0vec)
                issue_prefetch(jnp.minimum(i + 2, nch - 1), p)
            return _

        lax.fori_loop(1, (nch - 1) // 2, diter, None)
        body(nch - 1, 0, False, g=g, c0vec=c0vec)
        wait_ex(0)
        wait_ex(1)
        wait_prefetch(1)
        pltpu.sync_copy(den_v, denp_h.at[pl.ds((g * NW + wid) * N4P, N4P)])


def _l1r_body(denp_h, den_h, acc_v, tmp_v):
    wid = _worker_id()
    for g in range(2):
        off = wid * SEG1
        pltpu.sync_copy(denp_h.at[pl.ds(g * NW * N4P + off, SEG1)], acc_v)
        for t in range(1, NW):
            pltpu.sync_copy(denp_h.at[pl.ds((g * NW + t) * N4P + off, SEG1)], tmp_v)
            _vadd_loop(acc_v, tmp_v, 0, SEG1 // L)
        pltpu.sync_copy(acc_v, den_h.at[pl.ds(g * N4P + off, SEG1)])


def _copy_idx(dst_ref, src_ref):
    for k in range(K2 // L):
        dst_ref[pl.ds(k * L, L)] = src_ref[pl.ds(k * L, L)]


def _agg_pipeline(src_h, dst_h, wid, svs, dvs, dvsc, hbufs, obufs, out_sh,
                  sems, issue_ex, wait_ex, issue_h, wait_h, compute):
    ncheck = EW // K2
    assert ncheck % 2 == 1
    sem_sv, sem_dv, sem_s = sems

    def issue_prefetch(c, p):
        base = wid * EW + c * K2
        pltpu.async_copy(src_h.at[pl.ds(base, K2)], svs[p], sem_sv[p])
        pltpu.async_copy(dst_h.at[pl.ds(base, K2)], dvs[p], sem_dv[p])
        issue_ex(c, p)

    def wait_prefetch(p):
        pltpu.make_async_copy(src_h.at[pl.ds(0, K2)], svs[p], sem_sv[p]).wait()
        pltpu.make_async_copy(dst_h.at[pl.ds(0, K2)], dvs[p], sem_dv[p]).wait()
        wait_ex(p)

    def issue_scatter(p):
        pltpu.async_copy(obufs[p], out_sh.at[dvsc[p]], sem_s[p], add=True)

    def wait_scatter(p):
        pltpu.make_async_copy(obufs[p], out_sh.at[dvsc[p]], sem_s[p]).wait()

    issue_scatter(0)
    issue_scatter(1)
    issue_prefetch(0, 0)
    wait_prefetch(0)
    issue_h(0)
    issue_prefetch(1, 1)

    def diter(it, _):
        for p in range(2):
            i = it * 2 + p
            q = 1 - p
            wait_scatter(p)
            wait_prefetch(q)
            issue_h(q)
            wait_h(p)
            compute(p)
            _copy_idx(dvsc[p], dvs[p])
            issue_scatter(p)
            issue_prefetch(jnp.minimum(i + 2, ncheck - 1), p)
        return _

    lax.fori_loop(0, (ncheck - 1) // 2, diter, None)
    wait_scatter(0)
    wait_prefetch(1)
    wait_h(0)
    compute(0)
    _copy_idx(dvsc[0], dvs[0])
    issue_scatter(0)
    wait_scatter(0)
    wait_scatter(1)


def _l1p2_body(g, src_h, dst_h, ex_h, den_h, hg_h,
               outp_h,
               den_v, albuf,
               sv0, sv1, dv0, dv1, dvc0, dvc1, ex0, ex1,
               hb0, hb1, ob0, ob1,
               ssv0, ssv1, sdv0, sdv1, sex0, sex1, sh0, sh1, ss0, ss1,
               out_sh):
    cid = lax.axis_index("c")
    sid = lax.axis_index("s")
    wid = _worker_id()
    lidx = lax.iota(jnp.int32, L)
    svs, dvs, dvsc = (sv0, sv1), (dv0, dv1), (dvc0, dvc1)
    exs, hbufs, obufs = (ex0, ex1), (hb0, hb1), (ob0, ob1)
    sem_sv, sem_dv, sem_ex = (ssv0, ssv1), (sdv0, sdv1), (sex0, sex1)
    sem_h, sem_s = (sh0, sh1), (ss0, ss1)

    pltpu.sync_copy(den_h.at[pl.ds(g * N4P, N4P)], den_v)
    _zero2d(ob0, K2, 32)
    _zero2d(ob1, K2, 32)
    _zero_loop(dvc0, K2 // L, jnp.int32)
    _zero_loop(dvc1, K2 // L, jnp.int32)
    _zero_out_shared(out_sh, ob0, sid)
    plsc.subcore_barrier()

    def issue_ex(c, p):
        base = wid * EW + c * K2
        pltpu.async_copy(ex_h.at[pl.ds(g * EE * 4 + base * 4, K2 * 4)],
                         exs[p], sem_ex[p])

    def wait_ex(p):
        pltpu.make_async_copy(ex_h.at[pl.ds(0, K2 * 4)], exs[p],
                              sem_ex[p]).wait()

    def issue_h(p):
        pltpu.async_copy(hg_h.at[svs[p]], hbufs[p], sem_h[p])

    def wait_h(p):
        pltpu.make_async_copy(hg_h.at[svs[p]], hbufs[p], sem_h[p]).wait()

    def compute(p):
        exg_v, dv, hbuf, obuf = exs[p], dvs[p], hbufs[p], obufs[p]

        def astep(j, _):
            ex16 = exg_v[pl.ds(j * L, L)]
            d4 = plsc.load_gather(dv, [j * 4 + (lidx >> 2)])
            hdd = d4 * 4 + (lidx & 3)
            den16 = plsc.load_gather(den_v, [hdd])
            al = ex16 / (den16 + _EPS)
            aidx = (j * 4 + (lidx >> 2)) * 4 + (lidx & 3)
            plsc.store_scatter(albuf, [aidx], al)
            return _
        lax.fori_loop(0, K2 // 4, astep, None)

        def mstep(e, _):
            for q in range(2):
                av = plsc.load_gather(albuf, [e * 4 + q * 2 + (lidx >> 3)])
                obuf[e, pl.ds(q * L, L)] = av * hbuf[e, pl.ds(q * L, L)]
            return _
        lax.fori_loop(0, K2, mstep, None)

    _agg_pipeline(src_h, dst_h, wid, svs, dvs, dvsc, hbufs, obufs, out_sh,
                  (sem_sv, sem_dv, sem_s), issue_ex, wait_ex,
                  issue_h, wait_h, compute)
    plsc.subcore_barrier()
    _write_out_shared(out_sh, outp_h, cid, sid)


def _l2p1_body(src_h, dst_h, as_h, ad_h, c0_h,
               den_h, ex_h,
               as_v, ad_v, den_v, c0_v, sv, dv, exv, acc_v, tmp_v, den_sh):
    cid = lax.axis_index("c")
    sid = lax.axis_index("s")
    wid = _worker_id()
    pltpu.sync_copy(as_h, as_v)
    pltpu.sync_copy(ad_h, ad_v)
    pltpu.sync_copy(c0_h, c0_v)
    c0vec = c0_v[...]
    _zero_loop(den_v, NP // L)

    def chunk(i, _):
        base = wid * EW + i * K1
        pltpu.sync_copy(src_h.at[pl.ds(base, K1)], sv)
        pltpu.sync_copy(dst_h.at[pl.ds(base, K1)], dv)

        def step(j, _):
            s16 = sv[pl.ds(j * L, L)]
            d16 = dv[pl.ds(j * L, L)]
            a = plsc.load_gather(as_v, [s16])
            b = plsc.load_gather(ad_v, [d16])
            t = a + b
            e = jnp.where(t > 0, t, t * jnp.float32(0.2))
            ex = jnp.exp(e - c0vec)
            plsc.addupdate_scatter(den_v, [d16], ex)
            exv[pl.ds(j * L, L)] = ex
            return _

        lax.fori_loop(0, K1 // L, step, None)
        pltpu.sync_copy(exv, ex_h.at[pl.ds(base, K1)])
        return _

    lax.fori_loop(0, EW // K1, chunk, None)

    pltpu.sync_copy(den_v, den_sh.at[pl.ds(sid * NP, NP)])
    plsc.subcore_barrier()
    pltpu.sync_copy(den_sh.at[pl.ds(sid * SEG2, SEG2)], acc_v)
    for t in range(1, NS):
        pltpu.sync_copy(den_sh.at[pl.ds(t * NP + sid * SEG2, SEG2)], tmp_v)
        _vadd_loop(acc_v, tmp_v, 0, SEG2 // L)
    pltpu.sync_copy(acc_v, den_h.at[pl.ds(cid * NP + sid * SEG2, SEG2)])


def _l2p2_body(src_h, dst_h, ex_h, den_h, h2_h,
               outp_h,
               den_v, tmp_v, albuf,
               sv0, sv1, dv0, dv1, dvc0, dvc1, ex0, ex1,
               hb0, hb1, ob0, ob1,
               ssv0, ssv1, sdv0, sdv1, sex0, sex1, sh0, sh1, ss0, ss1,
               out_sh):
    cid = lax.axis_index("c")
    sid = lax.axis_index("s")
    wid = _worker_id()
    lidx = lax.iota(jnp.int32, L)
    svs, dvs, dvsc = (sv0, sv1), (dv0, dv1), (dvc0, dvc1)
    exs, hbufs, obufs = (ex0, ex1), (hb0, hb1), (ob0, ob1)
    sem_sv, sem_dv, sem_ex = (ssv0, ssv1), (sdv0, sdv1), (sex0, sex1)
    sem_h, sem_s = (sh0, sh1), (ss0, ss1)

    pltpu.sync_copy(den_h.at[pl.ds(0, NP)], den_v)
    for t in range(NS):
        pltpu.sync_copy(den_h.at[pl.ds(NP + t * SEG2, SEG2)], tmp_v)
        _vadd_loop(den_v, tmp_v, t * SEG2, SEG2 // L)

    _zero2d(ob0, K2, CP)
    _zero2d(ob1, K2, CP)
    _zero_loop(dvc0, K2 // L, jnp.int32)
    _zero_loop(dvc1, K2 // L, jnp.int32)
    _zero_out_shared(out_sh, ob0, sid)
    plsc.subcore_barrier()

    def issue_ex(c, p):
        base = wid * EW + c * K2
        pltpu.async_copy(ex_h.at[pl.ds(base, K2)], exs[p], sem_ex[p])

    def wait_ex(p):
        pltpu.make_async_copy(ex_h.at[pl.ds(0, K2)], exs[p], sem_ex[p]).wait()

    def issue_h(p):
        pltpu.async_copy(h2_h.at[svs[p]], hbufs[p], sem_h[p])

    def wait_h(p):
        pltpu.make_async_copy(h2_h.at[svs[p]], hbufs[p], sem_h[p]).wait()

    def compute(p):
        exv, dv, hbuf, obuf = exs[p], dvs[p], hbufs[p], obufs[p]

        def astep(j, _):
            d16 = dv[pl.ds(j * L, L)]
            den16 = plsc.load_gather(den_v, [d16])
            al = exv[pl.ds(j * L, L)] / (den16 + _EPS)
            albuf[pl.ds(j * L, L)] = al
            return _
        lax.fori_loop(0, K2 // L, astep, None)

        def mstep(e, _):
            av = plsc.load_gather(albuf, [lidx * 0 + e])
            for q in range(CP // L):
                obuf[e, pl.ds(q * L, L)] = av * hbuf[e, pl.ds(q * L, L)]
            return _
        lax.fori_loop(0, K2, mstep, None)

    _agg_pipeline(src_h, dst_h, wid, svs, dvs, dvsc, hbufs, obufs, out_sh,
                  (sem_sv, sem_dv, sem_s), issue_ex, wait_ex,
                  issue_h, wait_h, compute)
    plsc.subcore_barrier()
    _write_out_shared(out_sh, outp_h, cid, sid)


def _sc_mesh():
    return plsc.VectorSubcoreMesh(
        core_axis_name="c", subcore_axis_name="s", num_cores=NC, num_subcores=NS)


_SC_PARAMS = pltpu.CompilerParams(
    needs_layout_passes=False, use_tc_tiling_on_sc=False)


def kernel(x, edge_index, W1, att_src1, att_dst1, b1, W2, att_src2, att_dst2, b2):
    f32 = jnp.float32
    src = edge_index[0]
    dst = edge_index[1]

    asf = att_src1.reshape(64)
    adf = att_dst1.reshape(64)
    hm = (jnp.arange(64)[:, None] // 8 == jnp.arange(8)[None, :]).astype(f32)
    asmask = hm * asf[:, None]
    admask = hm * adf[:, None]

    h1, a_s, a_d, c0 = pl.pallas_call(
        _tc_a_body,
        out_shape=(
            jax.ShapeDtypeStruct((NN, 64), f32),
            jax.ShapeDtypeStruct((NN, 8), f32),
            jax.ShapeDtypeStruct((NN, 8), f32),
            jax.ShapeDtypeStruct((1, 8), f32),
        ),
    )(x, W1, asmask, admask)

    def to_groups(a):
        g = a.reshape(NN, 2, 4).transpose(1, 0, 2).reshape(2, NN * 4)
        return jnp.pad(g, ((0, 0), (0, N4P - NN * 4))).reshape(2 * N4P)

    asT = to_groups(a_s)
    adT = to_groups(a_d)
    c0dup = jnp.tile(c0.reshape(2, 4), (1, 4)).reshape(2 * L)
    h1a = h1[:, :32]
    h1b = h1[:, 32:]

    mesh = _sc_mesh()

    denp1, ex1 = pl.kernel(
        _l1p1_body, mesh=mesh, compiler_params=_SC_PARAMS,
        out_type=(
            jax.ShapeDtypeStruct((2 * NW * N4P,), f32),
            jax.ShapeDtypeStruct((2 * EE * 4,), f32),
        ),
        scratch_types=(
            [pltpu.VMEM((N4P,), f32)] * 3
            + [pltpu.VMEM((L,), f32)]
            + [pltpu.VMEM((K1,), jnp.int32)] * 4
            + [pltpu.VMEM((K1 * 4,), f32)] * 2
            + [pltpu.SemaphoreType.DMA] * 6
        ),
    )(src, dst, asT, adT, c0dup)

    den1 = pl.kernel(
        _l1r_body, mesh=mesh, compiler_params=_SC_PARAMS,
        out_type=jax.ShapeDtypeStruct((2 * N4P,), f32),
        scratch_types=[
            pltpu.VMEM((SEG1,), f32),
            pltpu.VMEM((SEG1,), f32),
        ],
    )(denp1)

    def l1p2(g, hg):
        return pl.kernel(
            functools.partial(_l1p2_body, g), mesh=mesh,
            compiler_params=_SC_PARAMS,
            out_type=jax.ShapeDtypeStruct((NC, NN, 32), f32),
            scratch_types=(
                [pltpu.VMEM((N4P,), f32), pltpu.VMEM((K2 * 4,), f32)]
                + [pltpu.VMEM((K2,), jnp.int32)] * 6
                + [pltpu.VMEM((K2 * 4,), f32)] * 2
                + [pltpu.VMEM((K2, 32), f32)] * 4
                + [pltpu.SemaphoreType.DMA] * 10
                + [pltpu.VMEM_SHARED((NN, 32), f32)]
            ),
        )(src, dst, ex1, den1, hg)

    outpa = l1p2(0, h1a)
    outpb = l1p2(1, h1b)

    w2p = jnp.pad(W2, ((0, 0), (0, CP - 40)))
    a2sp = jnp.pad(att_src2.reshape(40, 1), ((0, CP - 40), (0, 0)))
    a2dp = jnp.pad(att_dst2.reshape(40, 1), ((0, CP - 40), (0, 0)))
    emb, h2, as2, ad2, c02 = pl.pallas_call(
        _tc_b_body,
        out_shape=(
            jax.ShapeDtypeStruct((NN, 64), f32),
            jax.ShapeDtypeStruct((NN, CP), f32),
            jax.ShapeDtypeStruct((NN, 1), f32),
            jax.ShapeDtypeStruct((NN, 1), f32),
            jax.ShapeDtypeStruct((1, 1), f32),
        ),
    )(outpa, outpb, b1.reshape(1, 64), w2p, a2sp, a2dp)

    as2p = jnp.pad(as2.reshape(NN), (0, NP - NN))
    ad2p = jnp.pad(ad2.reshape(NN), (0, NP - NN))
    c02dup = jnp.broadcast_to(c02.reshape(1), (L,))

    den2, ex2 = pl.kernel(
        _l2p1_body, mesh=mesh, compiler_params=_SC_PARAMS,
        out_type=(
            jax.ShapeDtypeStruct((NC * NP,), f32),
            jax.ShapeDtypeStruct((EE,), f32),
        ),
        scratch_types=[
            pltpu.VMEM((NP,), f32),
            pltpu.VMEM((NP,), f32),
            pltpu.VMEM((NP,), f32),
            pltpu.VMEM((L,), f32),
            pltpu.VMEM((K1,), jnp.int32),
            pltpu.VMEM((K1,), jnp.int32),
            pltpu.VMEM((K1,), f32),
            pltpu.VMEM((SEG2,), f32),
            pltpu.VMEM((SEG2,), f32),
            pltpu.VMEM_SHARED((NS * NP,), f32),
        ],
    )(src, dst, as2p, ad2p, c02dup)

    outp2 = pl.kernel(
        _l2p2_body, mesh=mesh, compiler_params=_SC_PARAMS,
        out_type=jax.ShapeDtypeStruct((NC, NN, CP), f32),
        scratch_types=(
            [pltpu.VMEM((NP,), f32), pltpu.VMEM((SEG2,), f32),
             pltpu.VMEM((K2,), f32)]
            + [pltpu.VMEM((K2,), jnp.int32)] * 6
            + [pltpu.VMEM((K2,), f32)] * 2
            + [pltpu.VMEM((K2, CP), f32)] * 4
            + [pltpu.SemaphoreType.DMA] * 10
            + [pltpu.VMEM_SHARED((NN, CP), f32)]
        ),
    )(src, dst, ex2, den2, h2)

    b2p = jnp.pad(b2, (0, CP - 40)).reshape(1, CP)
    out48 = pl.pallas_call(
        _tc_c_body,
        out_shape=jax.ShapeDtypeStruct((NN, CP), f32),
    )(outp2, b2p)

    return (out48[:, :40], emb)


# speedup vs baseline: 52.3992x; 1.2019x over previous
import functools

import jax
import jax.numpy as jnp
from jax import lax
from jax.experimental import pallas as pl
from jax.experimental.pallas import tpu as pltpu
from jax.experimental.pallas import tpu_sc as plsc

NN = 10000
EE = 320000
NC = 2
NS = 16
NW = NC * NS
EW = EE // NW
L = 16

N4P = 40960
SEG1 = N4P // NW
NP = 10240
SEG2 = NP // NS
K1 = 400
K2 = 80
CP = 48
RT = 624
RTB = NS * RT
REX = NN - RTB

_EPS = 1e-16


def _tc_a_body(x_ref, w1_ref, asm_ref, adm_ref, h1_ref, as_ref, ad_ref, c0_ref):
    h1 = jnp.dot(x_ref[...], w1_ref[...], preferred_element_type=jnp.float32)
    h1_ref[...] = h1
    a_s = jnp.dot(h1, asm_ref[...], preferred_element_type=jnp.float32)
    a_d = jnp.dot(h1, adm_ref[...], preferred_element_type=jnp.float32)
    as_ref[...] = a_s
    ad_ref[...] = a_d
    c0 = jnp.max(a_s, axis=0, keepdims=True) + jnp.max(a_d, axis=0, keepdims=True)
    c0_ref[...] = jnp.maximum(c0, 0.0)


def _tc_b_body(pa_ref, pb_ref, den_ref, b1_ref, w2_ref, a2s_ref, a2d_ref,
               emb_ref, h2_ref, as2_ref, ad2_ref, c02_ref):
    ha = pa_ref[0] + pa_ref[1]
    hb = pb_ref[0] + pb_ref[1]
    h1 = (jnp.concatenate([ha, hb], axis=1) / (den_ref[...] + _EPS)
          + b1_ref[...])
    emb = jnp.where(h1 > 0, h1, jnp.exp(jnp.minimum(h1, 0.0)) - 1.0)
    emb_ref[...] = emb
    h2 = jnp.dot(emb, w2_ref[...], preferred_element_type=jnp.float32)
    h2_ref[...] = h2
    as2 = jnp.dot(h2, a2s_ref[...], preferred_element_type=jnp.float32)
    ad2 = jnp.dot(h2, a2d_ref[...], preferred_element_type=jnp.float32)
    as2_ref[...] = as2
    ad2_ref[...] = ad2
    c02 = jnp.max(as2, axis=0, keepdims=True) + jnp.max(ad2, axis=0, keepdims=True)
    c02_ref[...] = jnp.maximum(c02, 0.0)


def _tc_c_body(p_ref, d2_ref, b2_ref, out_ref):
    d2 = d2_ref[:, 0:1] + d2_ref[:, 1:2]
    o = (p_ref[0] + p_ref[1]) / (d2[:NN] + _EPS) + b2_ref[...]
    mask = lax.broadcasted_iota(jnp.int32, (NN, CP), 1) < 40
    xm = jnp.where(mask, o, jnp.float32(-1e30))
    m = jnp.max(xm, axis=1, keepdims=True)
    ex = jnp.where(mask, jnp.exp(o - m), 0.0)
    lse = jnp.log(jnp.sum(ex, axis=1, keepdims=True))
    out_ref[...] = o - m - lse


def _worker_id():
    return lax.axis_index("s") * NC + lax.axis_index("c")


def _vadd_loop(dst_ref, src_ref, dst_off, n_vregs):
    def body(i, _):
        o = dst_off + i * L
        dst_ref[pl.ds(o, L)] = dst_ref[pl.ds(o, L)] + src_ref[pl.ds(i * L, L)]
        return _
    lax.fori_loop(0, n_vregs, body, None)


def _zero_loop(dst_ref, n_vregs, dtype=jnp.float32):
    z = jnp.zeros((L,), dtype)
    def body(i, _):
        dst_ref[pl.ds(i * L, L)] = z
        return _
    lax.fori_loop(0, n_vregs, body, None)


def _zero2d(dst_ref, nrows, ncols):
    z = jnp.zeros((L,), jnp.float32)
    def body(r, _):
        for q in range(ncols // L):
            dst_ref[r, pl.ds(q * L, L)] = z
        return _
    lax.fori_loop(0, nrows, body, None)


def _zero_out_shared(out_sh, obuf, sid):
    stripe = sid * RT
    for r in range(0, RT - K2 + 1, K2):
        pltpu.sync_copy(obuf, out_sh.at[pl.ds(stripe + r, K2)])
    pltpu.sync_copy(obuf.at[pl.ds(0, RT - 7 * K2)],
                    out_sh.at[pl.ds(stripe + 7 * K2, RT - 7 * K2)])

    @pl.when(sid == 0)
    def _():
        pltpu.sync_copy(obuf.at[pl.ds(0, REX)], out_sh.at[pl.ds(RTB, REX)])


def _write_out_shared(out_sh, outp_h, cid, sid):
    stripe = sid * RT
    pltpu.sync_copy(out_sh.at[pl.ds(stripe, RT)],
                    outp_h.at[cid, pl.ds(stripe, RT)])

    @pl.when(sid == 0)
    def _():
        pltpu.sync_copy(out_sh.at[pl.ds(RTB, REX)],
                        outp_h.at[cid, pl.ds(RTB, REX)])


def _l1p1_body(src_h, dst_h, asT_h, adT_h, c0_h,
               denp_h, ex_h,
               as_v, ad_v, den_v, c0_v,
               sv0, sv1, dv0, dv1, exv0, exv1,
               ssv0, ssv1, sdv0, sdv1, sex0, sex1):
    wid = _worker_id()
    lidx = lax.iota(jnp.int32, L)
    svs, dvs, exvs = (sv0, sv1), (dv0, dv1), (exv0, exv1)
    sem_sv, sem_dv, sem_ex = (ssv0, ssv1), (sdv0, sdv1), (sex0, sex1)
    nch = EW // K1
    assert nch % 2 == 1

    def issue_prefetch(c, p):
        base = wid * EW + c * K1
        pltpu.async_copy(src_h.at[pl.ds(base, K1)], svs[p], sem_sv[p])
        pltpu.async_copy(dst_h.at[pl.ds(base, K1)], dvs[p], sem_dv[p])

    def wait_prefetch(p):
        pltpu.make_async_copy(src_h.at[pl.ds(0, K1)], svs[p], sem_sv[p]).wait()
        pltpu.make_async_copy(dst_h.at[pl.ds(0, K1)], dvs[p], sem_dv[p]).wait()

    def wait_ex(p):
        pltpu.make_async_copy(exv0, ex_h.at[pl.ds(0, K1 * 4)],
                              sem_ex[p]).wait()

    for g in range(2):
        pltpu.sync_copy(asT_h.at[pl.ds(g * N4P, N4P)], as_v)
        pltpu.sync_copy(adT_h.at[pl.ds(g * N4P, N4P)], ad_v)
        pltpu.sync_copy(c0_h.at[pl.ds(g * L, L)], c0_v)
        c0vec = c0_v[...]
        _zero_loop(den_v, N4P // L)

        def body(i, p, first, g=None, c0vec=None):
            sv, dv, exv = svs[p], dvs[p], exvs[p]
            if not first:
                wait_ex(p)
            wait_prefetch(p)

            def step(j, _):
                epos = j * 4 + (lidx >> 2)
                s4 = plsc.load_gather(sv, [epos])
                d4 = plsc.load_gather(dv, [epos])
                his = s4 * 4 + (lidx & 3)
                hdd = d4 * 4 + (lidx & 3)
                a = plsc.load_gather(as_v, [his])
                b = plsc.load_gather(ad_v, [hdd])
                t = a + b
                e = jnp.where(t > 0, t, t * jnp.float32(0.2))
                ex = jnp.exp(e - c0vec)
                plsc.addupdate_scatter(den_v, [hdd], ex)
                exv[pl.ds(j * L, L)] = ex
                return _

            lax.fori_loop(0, K1 // 4, step, None)
            base = wid * EW + i * K1
            pltpu.async_copy(exv, ex_h.at[pl.ds(g * EE * 4 + base * 4, K1 * 4)],
                             sem_ex[p])
            return base

        issue_prefetch(0, 0)
        issue_prefetch(1, 1)
        body(0, 0, True, g=g, c0vec=c0vec)
        issue_prefetch(2, 0)
        body(1, 1, True, g=g, c0vec=c0vec)
        issue_prefetch(3, 1)

        def diter(it, _, g=g, c0vec=c0vec):
            for p in range(2):
                i = it * 2 + p
                body(i, p, False, g=g, c0vec=c0vec)
                issue_prefetch(jnp.minimum(i + 2, nch - 1), p)
            return _

        lax.fori_loop(1, (nch - 1) // 2, diter, None)
        body(nch - 1, 0, False, g=g, c0vec=c0vec)
        wait_ex(0)
        wait_ex(1)
        wait_prefetch(1)
        pltpu.sync_copy(den_v, denp_h.at[pl.ds((g * NW + wid) * N4P, N4P)])


def _tc_r_body(denp_ref, den_ref):
    d = denp_ref[...]
    den_ref[0, :] = jnp.sum(d[:NW], axis=0)
    den_ref[1, :] = jnp.sum(d[NW:], axis=0)


def _copy_idx(dst_ref, src_ref):
    for k in range(K2 // L):
        dst_ref[pl.ds(k * L, L)] = src_ref[pl.ds(k * L, L)]


def _agg_pipeline(src_h, dst_h, wid, svs, dvs, dvsc, hbufs, obufs, out_sh,
                  sems, issue_ex, wait_ex, issue_h, wait_h, compute):
    ncheck = EW // K2
    assert ncheck % 2 == 1
    sem_sv, sem_dv, sem_s = sems

    def issue_prefetch(c, p):
        base = wid * EW + c * K2
        pltpu.async_copy(src_h.at[pl.ds(base, K2)], svs[p], sem_sv[p])
        pltpu.async_copy(dst_h.at[pl.ds(base, K2)], dvs[p], sem_dv[p])
        issue_ex(c, p)

    def wait_prefetch(p):
        pltpu.make_async_copy(src_h.at[pl.ds(0, K2)], svs[p], sem_sv[p]).wait()
        pltpu.make_async_copy(dst_h.at[pl.ds(0, K2)], dvs[p], sem_dv[p]).wait()
        wait_ex(p)

    def issue_scatter(p):
        pltpu.async_copy(obufs[p], out_sh.at[dvsc[p]], sem_s[p], add=True)

    def wait_scatter(p):
        pltpu.make_async_copy(obufs[p], out_sh.at[dvsc[p]], sem_s[p]).wait()

    issue_scatter(0)
    issue_scatter(1)
    issue_prefetch(0, 0)
    wait_prefetch(0)
    issue_h(0)
    issue_prefetch(1, 1)

    def diter(it, _):
        for p in range(2):
            i = it * 2 + p
            q = 1 - p
            wait_scatter(p)
            wait_prefetch(q)
            issue_h(q)
            wait_h(p)
            compute(p)
            _copy_idx(dvsc[p], dvs[p])
            issue_scatter(p)
            issue_prefetch(jnp.minimum(i + 2, ncheck - 1), p)
        return _

    lax.fori_loop(0, (ncheck - 1) // 2, diter, None)
    wait_scatter(0)
    wait_prefetch(1)
    wait_h(0)
    compute(0)
    _copy_idx(dvsc[0], dvs[0])
    issue_scatter(0)
    wait_scatter(0)
    wait_scatter(1)


def _l1p2_body(g, src_h, dst_h, ex_h, hg_h,
               outp_h,
               sv0, sv1, dv0, dv1, dvc0, dvc1, ex0, ex1,
               hb0, hb1, ob0, ob1,
               ssv0, ssv1, sdv0, sdv1, sex0, sex1, sh0, sh1, ss0, ss1,
               out_sh):
    cid = lax.axis_index("c")
    sid = lax.axis_index("s")
    wid = _worker_id()
    lidx = lax.iota(jnp.int32, L)
    svs, dvs, dvsc = (sv0, sv1), (dv0, dv1), (dvc0, dvc1)
    exs, hbufs, obufs = (ex0, ex1), (hb0, hb1), (ob0, ob1)
    sem_sv, sem_dv, sem_ex = (ssv0, ssv1), (sdv0, sdv1), (sex0, sex1)
    sem_h, sem_s = (sh0, sh1), (ss0, ss1)

    _zero2d(ob0, K2, 32)
    _zero2d(ob1, K2, 32)
    _zero_loop(dvc0, K2 // L, jnp.int32)
    _zero_loop(dvc1, K2 // L, jnp.int32)
    _zero_out_shared(out_sh, ob0, sid)
    plsc.subcore_barrier()

    def issue_ex(c, p):
        base = wid * EW + c * K2
        pltpu.async_copy(ex_h.at[pl.ds(g * EE * 4 + base * 4, K2 * 4)],
                         exs[p], sem_ex[p])

    def wait_ex(p):
        pltpu.make_async_copy(ex_h.at[pl.ds(0, K2 * 4)], exs[p],
                              sem_ex[p]).wait()

    def issue_h(p):
        pltpu.async_copy(hg_h.at[svs[p]], hbufs[p], sem_h[p])

    def wait_h(p):
        pltpu.make_async_copy(hg_h.at[svs[p]], hbufs[p], sem_h[p]).wait()

    def compute(p):
        exg_v, hbuf, obuf = exs[p], hbufs[p], obufs[p]

        def mstep(e, _):
            for q in range(2):
                av = plsc.load_gather(exg_v, [e * 4 + q * 2 + (lidx >> 3)])
                obuf[e, pl.ds(q * L, L)] = av * hbuf[e, pl.ds(q * L, L)]
            return _
        lax.fori_loop(0, K2, mstep, None)

    _agg_pipeline(src_h, dst_h, wid, svs, dvs, dvsc, hbufs, obufs, out_sh,
                  (sem_sv, sem_dv, sem_s), issue_ex, wait_ex,
                  issue_h, wait_h, compute)
    plsc.subcore_barrier()
    _write_out_shared(out_sh, outp_h, cid, sid)


def _l2p1_body(src_h, dst_h, as_h, ad_h, c0_h,
               den_h, ex_h,
               as_v, ad_v, den_v, c0_v, sv, dv, exv, acc_v, tmp_v, den_sh):
    cid = lax.axis_index("c")
    sid = lax.axis_index("s")
    wid = _worker_id()
    pltpu.sync_copy(as_h, as_v)
    pltpu.sync_copy(ad_h, ad_v)
    pltpu.sync_copy(c0_h, c0_v)
    c0vec = c0_v[...]
    _zero_loop(den_v, NP // L)

    def chunk(i, _):
        base = wid * EW + i * K1
        pltpu.sync_copy(src_h.at[pl.ds(base, K1)], sv)
        pltpu.sync_copy(dst_h.at[pl.ds(base, K1)], dv)

        def step(j, _):
            s16 = sv[pl.ds(j * L, L)]
            d16 = dv[pl.ds(j * L, L)]
            a = plsc.load_gather(as_v, [s16])
            b = plsc.load_gather(ad_v, [d16])
            t = a + b
            e = jnp.where(t > 0, t, t * jnp.float32(0.2))
            ex = jnp.exp(e - c0vec)
            plsc.addupdate_scatter(den_v, [d16], ex)
            exv[pl.ds(j * L, L)] = ex
            return _

        lax.fori_loop(0, K1 // L, step, None)
        pltpu.sync_copy(exv, ex_h.at[pl.ds(base, K1)])
        return _

    lax.fori_loop(0, EW // K1, chunk, None)

    pltpu.sync_copy(den_v, den_sh.at[pl.ds(sid * NP, NP)])
    plsc.subcore_barrier()
    pltpu.sync_copy(den_sh.at[pl.ds(sid * SEG2, SEG2)], acc_v)
    for t in range(1, NS):
        pltpu.sync_copy(den_sh.at[pl.ds(t * NP + sid * SEG2, SEG2)], tmp_v)
        _vadd_loop(acc_v, tmp_v, 0, SEG2 // L)
    pltpu.sync_copy(acc_v, den_h.at[pl.ds(cid * NP + sid * SEG2, SEG2)])


def _l2p2_body(src_h, dst_h, ex_h, h2_h,
               outp_h,
               sv0, sv1, dv0, dv1, dvc0, dvc1, ex0, ex1,
               hb0, hb1, ob0, ob1,
               ssv0, ssv1, sdv0, sdv1, sex0, sex1, sh0, sh1, ss0, ss1,
               out_sh):
    cid = lax.axis_index("c")
    sid = lax.axis_index("s")
    wid = _worker_id()
    lidx = lax.iota(jnp.int32, L)
    svs, dvs, dvsc = (sv0, sv1), (dv0, dv1), (dvc0, dvc1)
    exs, hbufs, obufs = (ex0, ex1), (hb0, hb1), (ob0, ob1)
    sem_sv, sem_dv, sem_ex = (ssv0, ssv1), (sdv0, sdv1), (sex0, sex1)
    sem_h, sem_s = (sh0, sh1), (ss0, ss1)

    _zero2d(ob0, K2, CP)
    _zero2d(ob1, K2, CP)
    _zero_loop(dvc0, K2 // L, jnp.int32)
    _zero_loop(dvc1, K2 // L, jnp.int32)
    _zero_out_shared(out_sh, ob0, sid)
    plsc.subcore_barrier()

    def issue_ex(c, p):
        base = wid * EW + c * K2
        pltpu.async_copy(ex_h.at[pl.ds(base, K2)], exs[p], sem_ex[p])

    def wait_ex(p):
        pltpu.make_async_copy(ex_h.at[pl.ds(0, K2)], exs[p], sem_ex[p]).wait()

    def issue_h(p):
        pltpu.async_copy(h2_h.at[svs[p]], hbufs[p], sem_h[p])

    def wait_h(p):
        pltpu.make_async_copy(h2_h.at[svs[p]], hbufs[p], sem_h[p]).wait()

    def compute(p):
        exv, hbuf, obuf = exs[p], hbufs[p], obufs[p]

        def mstep(e, _):
            av = plsc.load_gather(exv, [lidx * 0 + e])
            for q in range(CP // L):
                obuf[e, pl.ds(q * L, L)] = av * hbuf[e, pl.ds(q * L, L)]
            return _
        lax.fori_loop(0, K2, mstep, None)

    _agg_pipeline(src_h, dst_h, wid, svs, dvs, dvsc, hbufs, obufs, out_sh,
                  (sem_sv, sem_dv, sem_s), issue_ex, wait_ex,
                  issue_h, wait_h, compute)
    plsc.subcore_barrier()
    _write_out_shared(out_sh, outp_h, cid, sid)


def _sc_mesh():
    return plsc.VectorSubcoreMesh(
        core_axis_name="c", subcore_axis_name="s", num_cores=NC, num_subcores=NS)


_SC_PARAMS = pltpu.CompilerParams(
    needs_layout_passes=False, use_tc_tiling_on_sc=False)


def kernel(x, edge_index, W1, att_src1, att_dst1, b1, W2, att_src2, att_dst2, b2):
    f32 = jnp.float32
    src = edge_index[0]
    dst = edge_index[1]

    asf = att_src1.reshape(64)
    adf = att_dst1.reshape(64)
    hm = (jnp.arange(64)[:, None] // 8 == jnp.arange(8)[None, :]).astype(f32)
    asmask = hm * asf[:, None]
    admask = hm * adf[:, None]

    h1, a_s, a_d, c0 = pl.pallas_call(
        _tc_a_body,
        out_shape=(
            jax.ShapeDtypeStruct((NN, 64), f32),
            jax.ShapeDtypeStruct((NN, 8), f32),
            jax.ShapeDtypeStruct((NN, 8), f32),
            jax.ShapeDtypeStruct((1, 8), f32),
        ),
    )(x, W1, asmask, admask)

    def to_groups(a):
        g = a.reshape(NN, 2, 4).transpose(1, 0, 2).reshape(2, NN * 4)
        return jnp.pad(g, ((0, 0), (0, N4P - NN * 4))).reshape(2 * N4P)

    asT = to_groups(a_s)
    adT = to_groups(a_d)
    c0dup = jnp.tile(c0.reshape(2, 4), (1, 4)).reshape(2 * L)
    h1a = h1[:, :32]
    h1b = h1[:, 32:]

    mesh = _sc_mesh()

    denp1, ex1 = pl.kernel(
        _l1p1_body, mesh=mesh, compiler_params=_SC_PARAMS,
        out_type=(
            jax.ShapeDtypeStruct((2 * NW * N4P,), f32),
            jax.ShapeDtypeStruct((2 * EE * 4,), f32),
        ),
        scratch_types=(
            [pltpu.VMEM((N4P,), f32)] * 3
            + [pltpu.VMEM((L,), f32)]
            + [pltpu.VMEM((K1,), jnp.int32)] * 4
            + [pltpu.VMEM((K1 * 4,), f32)] * 2
            + [pltpu.SemaphoreType.DMA] * 6
        ),
    )(src, dst, asT, adT, c0dup)

    den1 = pl.pallas_call(
        _tc_r_body,
        out_shape=jax.ShapeDtypeStruct((2, N4P), f32),
    )(denp1.reshape(2 * NW, N4P))
    den64 = (den1[:, :NN * 4].reshape(2, NN, 4).transpose(1, 0, 2)
             .reshape(NN, 8).repeat(8, axis=1))

    def l1p2(g, hg):
        return pl.kernel(
            functools.partial(_l1p2_body, g), mesh=mesh,
            compiler_params=_SC_PARAMS,
            out_type=jax.ShapeDtypeStruct((NC, NN, 32), f32),
            scratch_types=(
                [pltpu.VMEM((K2,), jnp.int32)] * 6
                + [pltpu.VMEM((K2 * 4,), f32)] * 2
                + [pltpu.VMEM((K2, 32), f32)] * 4
                + [pltpu.SemaphoreType.DMA] * 10
                + [pltpu.VMEM_SHARED((NN, 32), f32)]
            ),
        )(src, dst, ex1, hg)

    outpa = l1p2(0, h1a)
    outpb = l1p2(1, h1b)

    w2p = jnp.pad(W2, ((0, 0), (0, CP - 40)))
    a2sp = jnp.pad(att_src2.reshape(40, 1), ((0, CP - 40), (0, 0)))
    a2dp = jnp.pad(att_dst2.reshape(40, 1), ((0, CP - 40), (0, 0)))
    emb, h2, as2, ad2, c02 = pl.pallas_call(
        _tc_b_body,
        out_shape=(
            jax.ShapeDtypeStruct((NN, 64), f32),
            jax.ShapeDtypeStruct((NN, CP), f32),
            jax.ShapeDtypeStruct((NN, 1), f32),
            jax.ShapeDtypeStruct((NN, 1), f32),
            jax.ShapeDtypeStruct((1, 1), f32),
        ),
    )(outpa, outpb, den64, b1.reshape(1, 64), w2p, a2sp, a2dp)

    as2p = jnp.pad(as2.reshape(NN), (0, NP - NN))
    ad2p = jnp.pad(ad2.reshape(NN), (0, NP - NN))
    c02dup = jnp.broadcast_to(c02.reshape(1), (L,))

    den2, ex2 = pl.kernel(
        _l2p1_body, mesh=mesh, compiler_params=_SC_PARAMS,
        out_type=(
            jax.ShapeDtypeStruct((NC * NP,), f32),
            jax.ShapeDtypeStruct((EE,), f32),
        ),
        scratch_types=[
            pltpu.VMEM((NP,), f32),
            pltpu.VMEM((NP,), f32),
            pltpu.VMEM((NP,), f32),
            pltpu.VMEM((L,), f32),
            pltpu.VMEM((K1,), jnp.int32),
            pltpu.VMEM((K1,), jnp.int32),
            pltpu.VMEM((K1,), f32),
            pltpu.VMEM((SEG2,), f32),
            pltpu.VMEM((SEG2,), f32),
            pltpu.VMEM_SHARED((NS * NP,), f32),
        ],
    )(src, dst, as2p, ad2p, c02dup)

    outp2 = pl.kernel(
        _l2p2_body, mesh=mesh, compiler_params=_SC_PARAMS,
        out_type=jax.ShapeDtypeStruct((NC, NN, CP), f32),
        scratch_types=(
            [pltpu.VMEM((K2,), jnp.int32)] * 6
            + [pltpu.VMEM((K2,), f32)] * 2
            + [pltpu.VMEM((K2, CP), f32)] * 4
            + [pltpu.SemaphoreType.DMA] * 10
            + [pltpu.VMEM_SHARED((NN, CP), f32)]
        ),
    )(src, dst, ex2, h2)

    b2p = jnp.pad(b2, (0, CP - 40)).reshape(1, CP)
    den2t = den2.reshape(NC, NP).transpose(1, 0)
    out48 = pl.pallas_call(
        _tc_c_body,
        out_shape=jax.ShapeDtypeStruct((NN, CP), f32),
    )(outp2, den2t, b2p)

    return (out48[:, :40], emb)


# speedup vs baseline: 53.1020x; 1.0134x over previous
import functools

import jax
import jax.numpy as jnp
from jax import lax
from jax.experimental import pallas as pl
from jax.experimental.pallas import tpu as pltpu
from jax.experimental.pallas import tpu_sc as plsc

NN = 10000
EE = 320000
NC = 2
NS = 16
NW = NC * NS
EW = EE // NW
L = 16

N4P = 40960
SEG1 = N4P // NW
NP = 10240
SEG2 = NP // NS
K1 = 400
K2 = 80
CP = 48
RT = 624
RTB = NS * RT
REX = NN - RTB

_EPS = 1e-16


def _tc_a_body(x_ref, w1_ref, asm_ref, adm_ref, h1_ref, as_ref, ad_ref, c0_ref):
    h1 = jnp.dot(x_ref[...], w1_ref[...], preferred_element_type=jnp.float32)
    h1_ref[...] = h1
    a_s = jnp.dot(h1, asm_ref[...], preferred_element_type=jnp.float32)
    a_d = jnp.dot(h1, adm_ref[...], preferred_element_type=jnp.float32)
    as_ref[...] = a_s
    ad_ref[...] = a_d
    c0 = jnp.max(a_s, axis=0, keepdims=True) + jnp.max(a_d, axis=0, keepdims=True)
    c0_ref[...] = jnp.maximum(c0, 0.0)


def _tc_b_body(pa_ref, pb_ref, den_ref, b1_ref, w2_ref, a2s_ref, a2d_ref,
               emb_ref, h2_ref, as2_ref, ad2_ref, c02_ref):
    ha = pa_ref[0] + pa_ref[1]
    hb = pb_ref[0] + pb_ref[1]
    h1 = (jnp.concatenate([ha, hb], axis=1) / (den_ref[...] + _EPS)
          + b1_ref[...])
    emb = jnp.where(h1 > 0, h1, jnp.exp(jnp.minimum(h1, 0.0)) - 1.0)
    emb_ref[...] = emb
    h2 = jnp.dot(emb, w2_ref[...], preferred_element_type=jnp.float32)
    h2_ref[...] = h2
    as2 = jnp.dot(h2, a2s_ref[...], preferred_element_type=jnp.float32)
    ad2 = jnp.dot(h2, a2d_ref[...], preferred_element_type=jnp.float32)
    as2_ref[...] = as2
    ad2_ref[...] = ad2
    c02 = jnp.max(as2, axis=0, keepdims=True) + jnp.max(ad2, axis=0, keepdims=True)
    c02_ref[...] = jnp.maximum(c02, 0.0)


def _tc_c_body(p_ref, d2_ref, b2_ref, out_ref):
    d2 = d2_ref[:, 0:1] + d2_ref[:, 1:2]
    o = (p_ref[0] + p_ref[1]) / (d2[:NN] + _EPS) + b2_ref[...]
    mask = lax.broadcasted_iota(jnp.int32, (NN, CP), 1) < 40
    xm = jnp.where(mask, o, jnp.float32(-1e30))
    m = jnp.max(xm, axis=1, keepdims=True)
    ex = jnp.where(mask, jnp.exp(o - m), 0.0)
    lse = jnp.log(jnp.sum(ex, axis=1, keepdims=True))
    out_ref[...] = o - m - lse


def _worker_id():
    return lax.axis_index("s") * NC + lax.axis_index("c")


def _vadd_loop(dst_ref, src_ref, dst_off, n_vregs):
    def body(i, _):
        o = dst_off + i * L
        dst_ref[pl.ds(o, L)] = dst_ref[pl.ds(o, L)] + src_ref[pl.ds(i * L, L)]
        return _
    lax.fori_loop(0, n_vregs, body, None)


def _zero_loop(dst_ref, n_vregs, dtype=jnp.float32):
    z = jnp.zeros((L,), dtype)
    def body(i, _):
        dst_ref[pl.ds(i * L, L)] = z
        return _
    lax.fori_loop(0, n_vregs, body, None)


def _zero2d(dst_ref, nrows, ncols):
    z = jnp.zeros((L,), jnp.float32)
    def body(r, _):
        for q in range(ncols // L):
            dst_ref[r, pl.ds(q * L, L)] = z
        return _
    lax.fori_loop(0, nrows, body, None)


def _zero_out_shared(out_sh, obuf, sid):
    stripe = sid * RT
    for r in range(0, RT - K2 + 1, K2):
        pltpu.sync_copy(obuf, out_sh.at[pl.ds(stripe + r, K2)])
    pltpu.sync_copy(obuf.at[pl.ds(0, RT - 7 * K2)],
                    out_sh.at[pl.ds(stripe + 7 * K2, RT - 7 * K2)])

    @pl.when(sid == 0)
    def _():
        pltpu.sync_copy(obuf.at[pl.ds(0, REX)], out_sh.at[pl.ds(RTB, REX)])


def _write_out_shared(out_sh, outp_h, cid, sid):
    stripe = sid * RT
    pltpu.sync_copy(out_sh.at[pl.ds(stripe, RT)],
                    outp_h.at[cid, pl.ds(stripe, RT)])

    @pl.when(sid == 0)
    def _():
        pltpu.sync_copy(out_sh.at[pl.ds(RTB, REX)],
                        outp_h.at[cid, pl.ds(RTB, REX)])


def _l1p1_body(src_h, dst_h, asT_h, adT_h, c0_h,
               denp_h, ex_h,
               as_v, ad_v, den_v, c0_v,
               sv0, sv1, dv0, dv1, exv0, exv1,
               ssv0, ssv1, sdv0, sdv1, sex0, sex1):
    wid = _worker_id()
    lidx = lax.iota(jnp.int32, L)
    svs, dvs, exvs = (sv0, sv1), (dv0, dv1), (exv0, exv1)
    sem_sv, sem_dv, sem_ex = (ssv0, ssv1), (sdv0, sdv1), (sex0, sex1)
    nch = EW // K1
    assert nch % 2 == 1

    def issue_prefetch(c, p):
        base = wid * EW + c * K1
        pltpu.async_copy(src_h.at[pl.ds(base, K1)], svs[p], sem_sv[p])
        pltpu.async_copy(dst_h.at[pl.ds(base, K1)], dvs[p], sem_dv[p])

    def wait_prefetch(p):
        pltpu.make_async_copy(src_h.at[pl.ds(0, K1)], svs[p], sem_sv[p]).wait()
        pltpu.make_async_copy(dst_h.at[pl.ds(0, K1)], dvs[p], sem_dv[p]).wait()

    def wait_ex(p):
        pltpu.make_async_copy(exv0, ex_h.at[pl.ds(0, K1 * 4)],
                              sem_ex[p]).wait()

    for g in range(2):
        pltpu.sync_copy(asT_h.at[pl.ds(g * N4P, N4P)], as_v)
        pltpu.sync_copy(adT_h.at[pl.ds(g * N4P, N4P)], ad_v)
        pltpu.sync_copy(c0_h.at[pl.ds(g * L, L)], c0_v)
        c0vec = c0_v[...]
        _zero_loop(den_v, N4P // L)

        def body(i, p, first, g=None, c0vec=None):
            sv, dv, exv = svs[p], dvs[p], exvs[p]
            if not first:
                wait_ex(p)
            wait_prefetch(p)

            def step(j, _):
                epos = j * 4 + (lidx >> 2)
                s4 = plsc.load_gather(sv, [epos])
                d4 = plsc.load_gather(dv, [epos])
                his = s4 * 4 + (lidx & 3)
                hdd = d4 * 4 + (lidx & 3)
                a = plsc.load_gather(as_v, [his])
                b = plsc.load_gather(ad_v, [hdd])
                t = a + b
                e = jnp.where(t > 0, t, t * jnp.float32(0.2))
                ex = jnp.exp(e - c0vec)
                plsc.addupdate_scatter(den_v, [hdd], ex)
                exv[pl.ds(j * L, L)] = ex
                return _

            lax.fori_loop(0, K1 // 4, step, None)
            base = wid * EW + i * K1
            pltpu.async_copy(exv, ex_h.at[pl.ds(g * EE * 4 + base * 4, K1 * 4)],
                             sem_ex[p])
            return base

        issue_prefetch(0, 0)
        issue_prefetch(1, 1)
        body(0, 0, True, g=g, c0vec=c0vec)
        issue_prefetch(2, 0)
        body(1, 1, True, g=g, c0vec=c0vec)
        issue_prefetch(3, 1)

        def diter(it, _, g=g, c0vec=c0vec):
            for p in range(2):
                i = it * 2 + p
                body(i, p, False, g=g, c0vec=c0vec)
                issue_prefetch(jnp.minimum(i + 2, nch - 1), p)
            return _

        lax.fori_loop(1, (nch - 1) // 2, diter, None)
        body(nch - 1, 0, False, g=g, c0vec=c0vec)
        wait_ex(0)
        wait_ex(1)
        wait_prefetch(1)
        pltpu.sync_copy(den_v, denp_h.at[pl.ds((g * NW + wid) * N4P, N4P)])


def _tc_r_body(denp_ref, den_ref):
    d = denp_ref[...]
    den_ref[0, :] = jnp.sum(d[:NW], axis=0)
    den_ref[1, :] = jnp.sum(d[NW:], axis=0)


def _copy_idx(dst_ref, src_ref):
    for k in range(K2 // L):
        dst_ref[pl.ds(k * L, L)] = src_ref[pl.ds(k * L, L)]


def _agg_pipeline(src_h, dst_h, wid, svs, dvs, dvsc, hbufs, obufs, out_sh,
                  sems, issue_ex, wait_ex, issue_h, wait_h, compute):
    ncheck = EW // K2
    assert ncheck % 2 == 1
    sem_sv, sem_dv, sem_s = sems

    def issue_prefetch(c, p):
        base = wid * EW + c * K2
        pltpu.async_copy(src_h.at[pl.ds(base, K2)], svs[p], sem_sv[p])
        pltpu.async_copy(dst_h.at[pl.ds(base, K2)], dvs[p], sem_dv[p])
        issue_ex(c, p)

    def wait_prefetch(p):
        pltpu.make_async_copy(src_h.at[pl.ds(0, K2)], svs[p], sem_sv[p]).wait()
        pltpu.make_async_copy(dst_h.at[pl.ds(0, K2)], dvs[p], sem_dv[p]).wait()
        wait_ex(p)

    def issue_scatter(p):
        pltpu.async_copy(obufs[p], out_sh.at[dvsc[p]], sem_s[p], add=True)

    def wait_scatter(p):
        pltpu.make_async_copy(obufs[p], out_sh.at[dvsc[p]], sem_s[p]).wait()

    issue_scatter(0)
    issue_scatter(1)
    issue_prefetch(0, 0)
    wait_prefetch(0)
    issue_h(0)
    issue_prefetch(1, 1)

    def diter(it, _):
        for p in range(2):
            i = it * 2 + p
            q = 1 - p
            wait_scatter(p)
            wait_prefetch(q)
            issue_h(q)
            wait_h(p)
            compute(p)
            _copy_idx(dvsc[p], dvs[p])
            issue_scatter(p)
            issue_prefetch(jnp.minimum(i + 2, ncheck - 1), p)
        return _

    lax.fori_loop(0, (ncheck - 1) // 2, diter, None)
    wait_scatter(0)
    wait_prefetch(1)
    wait_h(0)
    compute(0)
    _copy_idx(dvsc[0], dvs[0])
    issue_scatter(0)
    wait_scatter(0)
    wait_scatter(1)


def _l1p2_body(g, src_h, dst_h, ex_h, hg_h,
               outp_h,
               sv0, sv1, dv0, dv1, dvc0, dvc1, ex0, ex1,
               hb0, hb1, ob0, ob1,
               ssv0, ssv1, sdv0, sdv1, sex0, sex1, sh0, sh1, ss0, ss1,
               out_sh):
    cid = lax.axis_index("c")
    sid = lax.axis_index("s")
    wid = _worker_id()
    lidx = lax.iota(jnp.int32, L)
    svs, dvs, dvsc = (sv0, sv1), (dv0, dv1), (dvc0, dvc1)
    exs, hbufs, obufs = (ex0, ex1), (hb0, hb1), (ob0, ob1)
    sem_sv, sem_dv, sem_ex = (ssv0, ssv1), (sdv0, sdv1), (sex0, sex1)
    sem_h, sem_s = (sh0, sh1), (ss0, ss1)

    _zero2d(ob0, K2, 32)
    _zero2d(ob1, K2, 32)
    _zero_loop(dvc0, K2 // L, jnp.int32)
    _zero_loop(dvc1, K2 // L, jnp.int32)
    _zero_out_shared(out_sh, ob0, sid)
    plsc.subcore_barrier()

    def issue_ex(c, p):
        base = wid * EW + c * K2
        pltpu.async_copy(ex_h.at[pl.ds(g * EE * 4 + base * 4, K2 * 4)],
                         exs[p], sem_ex[p])

    def wait_ex(p):
        pltpu.make_async_copy(ex_h.at[pl.ds(0, K2 * 4)], exs[p],
                              sem_ex[p]).wait()

    def issue_h(p):
        pltpu.async_copy(hg_h.at[svs[p]], hbufs[p], sem_h[p])

    def wait_h(p):
        pltpu.make_async_copy(hg_h.at[svs[p]], hbufs[p], sem_h[p]).wait()

    def compute(p):
        exg_v, hbuf, obuf = exs[p], hbufs[p], obufs[p]

        def mstep(i, _):
            for u in range(2):
                e = i * 2 + u
                for q in range(2):
                    av = plsc.load_gather(exg_v,
                                          [e * 4 + q * 2 + (lidx >> 3)])
                    obuf[e, pl.ds(q * L, L)] = av * hbuf[e, pl.ds(q * L, L)]
            return _
        lax.fori_loop(0, K2 // 2, mstep, None)

    _agg_pipeline(src_h, dst_h, wid, svs, dvs, dvsc, hbufs, obufs, out_sh,
                  (sem_sv, sem_dv, sem_s), issue_ex, wait_ex,
                  issue_h, wait_h, compute)
    plsc.subcore_barrier()
    _write_out_shared(out_sh, outp_h, cid, sid)


def _l2p1_body(src_h, dst_h, as_h, ad_h, c0_h,
               den_h, ex_h,
               as_v, ad_v, den_v, c0_v, sv, dv, exv, acc_v, tmp_v, den_sh):
    cid = lax.axis_index("c")
    sid = lax.axis_index("s")
    wid = _worker_id()
    pltpu.sync_copy(as_h, as_v)
    pltpu.sync_copy(ad_h, ad_v)
    pltpu.sync_copy(c0_h, c0_v)
    c0vec = c0_v[...]
    _zero_loop(den_v, NP // L)

    def chunk(i, _):
        base = wid * EW + i * K1
        pltpu.sync_copy(src_h.at[pl.ds(base, K1)], sv)
        pltpu.sync_copy(dst_h.at[pl.ds(base, K1)], dv)

        def step(j, _):
            s16 = sv[pl.ds(j * L, L)]
            d16 = dv[pl.ds(j * L, L)]
            a = plsc.load_gather(as_v, [s16])
            b = plsc.load_gather(ad_v, [d16])
            t = a + b
            e = jnp.where(t > 0, t, t * jnp.float32(0.2))
            ex = jnp.exp(e - c0vec)
            plsc.addupdate_scatter(den_v, [d16], ex)
            exv[pl.ds(j * L, L)] = ex
            return _

        lax.fori_loop(0, K1 // L, step, None)
        pltpu.sync_copy(exv, ex_h.at[pl.ds(base, K1)])
        return _

    lax.fori_loop(0, EW // K1, chunk, None)

    pltpu.sync_copy(den_v, den_sh.at[pl.ds(sid * NP, NP)])
    plsc.subcore_barrier()
    pltpu.sync_copy(den_sh.at[pl.ds(sid * SEG2, SEG2)], acc_v)
    for t in range(1, NS):
        pltpu.sync_copy(den_sh.at[pl.ds(t * NP + sid * SEG2, SEG2)], tmp_v)
        _vadd_loop(acc_v, tmp_v, 0, SEG2 // L)
    pltpu.sync_copy(acc_v, den_h.at[pl.ds(cid * NP + sid * SEG2, SEG2)])


def _l2p2_body(src_h, dst_h, ex_h, h2_h,
               outp_h,
               sv0, sv1, dv0, dv1, dvc0, dvc1, ex0, ex1,
               hb0, hb1, ob0, ob1,
               ssv0, ssv1, sdv0, sdv1, sex0, sex1, sh0, sh1, ss0, ss1,
               out_sh):
    cid = lax.axis_index("c")
    sid = lax.axis_index("s")
    wid = _worker_id()
    lidx = lax.iota(jnp.int32, L)
    svs, dvs, dvsc = (sv0, sv1), (dv0, dv1), (dvc0, dvc1)
    exs, hbufs, obufs = (ex0, ex1), (hb0, hb1), (ob0, ob1)
    sem_sv, sem_dv, sem_ex = (ssv0, ssv1), (sdv0, sdv1), (sex0, sex1)
    sem_h, sem_s = (sh0, sh1), (ss0, ss1)

    _zero2d(ob0, K2, CP)
    _zero2d(ob1, K2, CP)
    _zero_loop(dvc0, K2 // L, jnp.int32)
    _zero_loop(dvc1, K2 // L, jnp.int32)
    _zero_out_shared(out_sh, ob0, sid)
    plsc.subcore_barrier()

    def issue_ex(c, p):
        base = wid * EW + c * K2
        pltpu.async_copy(ex_h.at[pl.ds(base, K2)], exs[p], sem_ex[p])

    def wait_ex(p):
        pltpu.make_async_copy(ex_h.at[pl.ds(0, K2)], exs[p], sem_ex[p]).wait()

    def issue_h(p):
        pltpu.async_copy(h2_h.at[svs[p]], hbufs[p], sem_h[p])

    def wait_h(p):
        pltpu.make_async_copy(h2_h.at[svs[p]], hbufs[p], sem_h[p]).wait()

    def compute(p):
        exv, hbuf, obuf = exs[p], hbufs[p], obufs[p]

        def mstep(i, _):
            for u in range(2):
                e = i * 2 + u
                av = plsc.load_gather(exv, [lidx * 0 + e])
                for q in range(CP // L):
                    obuf[e, pl.ds(q * L, L)] = av * hbuf[e, pl.ds(q * L, L)]
            return _
        lax.fori_loop(0, K2 // 2, mstep, None)

    _agg_pipeline(src_h, dst_h, wid, svs, dvs, dvsc, hbufs, obufs, out_sh,
                  (sem_sv, sem_dv, sem_s), issue_ex, wait_ex,
                  issue_h, wait_h, compute)
    plsc.subcore_barrier()
    _write_out_shared(out_sh, outp_h, cid, sid)


def _sc_mesh():
    return plsc.VectorSubcoreMesh(
        core_axis_name="c", subcore_axis_name="s", num_cores=NC, num_subcores=NS)


_SC_PARAMS = pltpu.CompilerParams(
    needs_layout_passes=False, use_tc_tiling_on_sc=False)


def kernel(x, edge_index, W1, att_src1, att_dst1, b1, W2, att_src2, att_dst2, b2):
    f32 = jnp.float32
    src = edge_index[0]
    dst = edge_index[1]

    asf = att_src1.reshape(64)
    adf = att_dst1.reshape(64)
    hm = (jnp.arange(64)[:, None] // 8 == jnp.arange(8)[None, :]).astype(f32)
    asmask = hm * asf[:, None]
    admask = hm * adf[:, None]

    h1, a_s, a_d, c0 = pl.pallas_call(
        _tc_a_body,
        out_shape=(
            jax.ShapeDtypeStruct((NN, 64), f32),
            jax.ShapeDtypeStruct((NN, 8), f32),
            jax.ShapeDtypeStruct((NN, 8), f32),
            jax.ShapeDtypeStruct((1, 8), f32),
        ),
    )(x, W1, asmask, admask)

    def to_groups(a):
        g = a.reshape(NN, 2, 4).transpose(1, 0, 2).reshape(2, NN * 4)
        return jnp.pad(g, ((0, 0), (0, N4P - NN * 4))).reshape(2 * N4P)

    asT = to_groups(a_s)
    adT = to_groups(a_d)
    c0dup = jnp.tile(c0.reshape(2, 4), (1, 4)).reshape(2 * L)
    h1a = h1[:, :32]
    h1b = h1[:, 32:]

    mesh = _sc_mesh()

    denp1, ex1 = pl.kernel(
        _l1p1_body, mesh=mesh, compiler_params=_SC_PARAMS,
        out_type=(
            jax.ShapeDtypeStruct((2 * NW * N4P,), f32),
            jax.ShapeDtypeStruct((2 * EE * 4,), f32),
        ),
        scratch_types=(
            [pltpu.VMEM((N4P,), f32)] * 3
            + [pltpu.VMEM((L,), f32)]
            + [pltpu.VMEM((K1,), jnp.int32)] * 4
            + [pltpu.VMEM((K1 * 4,), f32)] * 2
            + [pltpu.SemaphoreType.DMA] * 6
        ),
    )(src, dst, asT, adT, c0dup)

    den1 = pl.pallas_call(
        _tc_r_body,
        out_shape=jax.ShapeDtypeStruct((2, N4P), f32),
    )(denp1.reshape(2 * NW, N4P))
    den64 = (den1[:, :NN * 4].reshape(2, NN, 4).transpose(1, 0, 2)
             .reshape(NN, 8).repeat(8, axis=1))

    def l1p2(g, hg):
        return pl.kernel(
            functools.partial(_l1p2_body, g), mesh=mesh,
            compiler_params=_SC_PARAMS,
            out_type=jax.ShapeDtypeStruct((NC, NN, 32), f32),
            scratch_types=(
                [pltpu.VMEM((K2,), jnp.int32)] * 6
                + [pltpu.VMEM((K2 * 4,), f32)] * 2
                + [pltpu.VMEM((K2, 32), f32)] * 4
                + [pltpu.SemaphoreType.DMA] * 10
                + [pltpu.VMEM_SHARED((NN, 32), f32)]
            ),
        )(src, dst, ex1, hg)

    outpa = l1p2(0, h1a)
    outpb = l1p2(1, h1b)

    w2p = jnp.pad(W2, ((0, 0), (0, CP - 40)))
    a2sp = jnp.pad(att_src2.reshape(40, 1), ((0, CP - 40), (0, 0)))
    a2dp = jnp.pad(att_dst2.reshape(40, 1), ((0, CP - 40), (0, 0)))
    emb, h2, as2, ad2, c02 = pl.pallas_call(
        _tc_b_body,
        out_shape=(
            jax.ShapeDtypeStruct((NN, 64), f32),
            jax.ShapeDtypeStruct((NN, CP), f32),
            jax.ShapeDtypeStruct((NN, 1), f32),
            jax.ShapeDtypeStruct((NN, 1), f32),
            jax.ShapeDtypeStruct((1, 1), f32),
        ),
    )(outpa, outpb, den64, b1.reshape(1, 64), w2p, a2sp, a2dp)

    as2p = jnp.pad(as2.reshape(NN), (0, NP - NN))
    ad2p = jnp.pad(ad2.reshape(NN), (0, NP - NN))
    c02dup = jnp.broadcast_to(c02.reshape(1), (L,))

    den2, ex2 = pl.kernel(
        _l2p1_body, mesh=mesh, compiler_params=_SC_PARAMS,
        out_type=(
            jax.ShapeDtypeStruct((NC * NP,), f32),
            jax.ShapeDtypeStruct((EE,), f32),
        ),
        scratch_types=[
            pltpu.VMEM((NP,), f32),
            pltpu.VMEM((NP,), f32),
            pltpu.VMEM((NP,), f32),
            pltpu.VMEM((L,), f32),
            pltpu.VMEM((K1,), jnp.int32),
            pltpu.VMEM((K1,), jnp.int32),
            pltpu.VMEM((K1,), f32),
            pltpu.VMEM((SEG2,), f32),
            pltpu.VMEM((SEG2,), f32),
            pltpu.VMEM_SHARED((NS * NP,), f32),
        ],
    )(src, dst, as2p, ad2p, c02dup)

    outp2 = pl.kernel(
        _l2p2_body, mesh=mesh, compiler_params=_SC_PARAMS,
        out_type=jax.ShapeDtypeStruct((NC, NN, CP), f32),
        scratch_types=(
            [pltpu.VMEM((K2,), jnp.int32)] * 6
            + [pltpu.VMEM((K2,), f32)] * 2
            + [pltpu.VMEM((K2, CP), f32)] * 4
            + [pltpu.SemaphoreType.DMA] * 10
            + [pltpu.VMEM_SHARED((NN, CP), f32)]
        ),
    )(src, dst, ex2, h2)

    b2p = jnp.pad(b2, (0, CP - 40)).reshape(1, CP)
    den2t = den2.reshape(NC, NP).transpose(1, 0)
    out48 = pl.pallas_call(
        _tc_c_body,
        out_shape=jax.ShapeDtypeStruct((NN, CP), f32),
    )(outp2, den2t, b2p)

    return (out48[:, :40], emb)
